# baseline (speedup 1.0000x reference)
import functools

import jax
import jax.numpy as jnp
from jax import lax
from jax.experimental import pallas as pl
from jax.experimental.pallas import tpu as pltpu
from jax.experimental.pallas import tpu_sc as plsc

NN = 4096
EE = 65536
NC, NS, L = 2, 16, 16

f32 = jnp.float32
i32 = jnp.int32


def _sc_mesh():
    return plsc.VectorSubcoreMesh(core_axis_name="c", subcore_axis_name="s")


def _zero_vmem_rows(ref, nrows, width):
    def body(e, _):
        for g in range(width // L):
            ref[e, pl.ds(g * L, L)] = jnp.zeros((L,), f32)
        return 0
    lax.fori_loop(0, nrows, body, 0)


def _zero_vmem_1d(ref, n):
    def body(e, _):
        ref[pl.ds(e * L, L)] = jnp.zeros((L,), f32)
        return 0
    lax.fori_loop(0, n // L, body, 0)


def _sc_degrees(edst, esrc, sdst, ssrc, w):
    EPS = EE // (NC * NS)
    CH = 128
    NCHK = EPS // CH
    AW = 4 * NN

    def body(edst_h, esrc_h, sdst_h, ssrc_h, w_h, out_h,
             acc, dstv, srcv, sdv, ssv, wv, idxb, ones_b, zv, drb):
        cid = lax.axis_index("c")
        sid = lax.axis_index("s")
        base = (cid * NS + sid) * EPS
        pltpu.sync_copy(edst_h.at[pl.ds(base, EPS)], dstv)
        pltpu.sync_copy(esrc_h.at[pl.ds(base, EPS)], srcv)
        pltpu.sync_copy(sdst_h.at[pl.ds(base, EPS)], sdv)
        pltpu.sync_copy(ssrc_h.at[pl.ds(base, EPS)], ssv)
        pltpu.sync_copy(w_h.at[pl.ds(base, EPS)], wv)
        _zero_vmem_1d(zv, 1024)

        def fill_ones(e, _):
            ones_b[pl.ds(e * L, L)] = jnp.full((L,), 1.0, f32)
            return 0
        lax.fori_loop(0, CH // L, fill_ones, 0)

        pltpu.sync_copy(zv, acc.at[pl.ds(sid * 1024, 1024)])
        plsc.subcore_barrier()

        for c in range(NCHK):
            for (vec, off, vals) in (
                    (dstv, 0, ones_b), (srcv, NN, ones_b),
                    (sdv, 2 * NN, wv.at[pl.ds(c * CH, CH)]),
                    (ssv, 3 * NN, wv.at[pl.ds(c * CH, CH)])):
                for g in range(CH // L):
                    v = vec[pl.ds(c * CH + g * L, L)]
                    idxb[pl.ds(g * L, L)] = v + off
                pltpu.sync_copy(vals, acc.at[idxb], add=True)
        pltpu.sync_copy(zv.at[pl.ds(0, CH)], acc.at[idxb], add=True)
        pltpu.sync_copy(acc.at[pl.ds(0, L)], drb)
        plsc.subcore_barrier()
        pltpu.sync_copy(acc.at[pl.ds(sid * 1024, 1024)],
                        out_h.at[cid, pl.ds(sid * 1024, 1024)])

    fn = pl.kernel(
        body,
        out_type=jax.ShapeDtypeStruct((NC, AW), f32),
        mesh=_sc_mesh(),
        compiler_params=pltpu.CompilerParams(use_tc_tiling_on_sc=False),
        scratch_types=[
            pltpu.VMEM_SHARED((AW,), f32),
            pltpu.VMEM((EPS,), i32), pltpu.VMEM((EPS,), i32),
            pltpu.VMEM((EPS,), i32), pltpu.VMEM((EPS,), i32),
            pltpu.VMEM((EPS,), f32),
            pltpu.VMEM((CH,), i32),
            pltpu.VMEM((CH,), f32),
            pltpu.VMEM((1024,), f32),
            pltpu.VMEM((L,), f32),
        ],
    )
    return fn(edst, esrc, sdst, ssrc, w)


def _sc_build_adj(edst, esrc, tok):
    EPS = EE // NS
    CH = 128
    NCHK = EPS // CH
    SLABW = 256 * NN
    NSLAB = (NN * NN) // SLABW
    ZW = 16384

    def body(edst_h, esrc_h, tok_h, out_h,
             acc, dstv, srcv, idxb, ones_b, mns_b, zv, drb):
        cid = lax.axis_index("c")
        sid = lax.axis_index("s")
        base = sid * EPS
        pltpu.sync_copy(tok_h, drb)
        pltpu.sync_copy(edst_h.at[pl.ds(base, EPS)], dstv)
        pltpu.sync_copy(esrc_h.at[pl.ds(base, EPS)], srcv)
        _zero_vmem_1d(zv, ZW)

        def fill_ones(e, _):
            ones_b[pl.ds(e * L, L)] = jnp.full((L,), 1.0, f32)
            mns_b[pl.ds(e * L, L)] = jnp.full((L,), -1.0, f32)
            return 0
        lax.fori_loop(0, CH // L, fill_ones, 0)

        for z in range(SLABW // NS // ZW):
            pltpu.sync_copy(zv, acc.at[pl.ds(sid * (SLABW // NS) + z * ZW, ZW)])
        @pl.when(sid == 0)
        def _():
            pltpu.sync_copy(zv.at[pl.ds(0, L)], acc.at[pl.ds(SLABW, L)])
        plsc.subcore_barrier()

        def do_slab(t, _):
            slab = 2 * t + cid
            word0 = slab * SLABW

            def scatter_pass(vals):
                def do_chunk(c, _):
                    for g in range(CH // L):
                        d = dstv[pl.ds(c * CH + g * L, L)]
                        s = srcv[pl.ds(c * CH + g * L, L)]
                        loc = d * NN + s - word0
                        ok = (loc >= 0) & (loc < SLABW)
                        idxb[pl.ds(g * L, L)] = jnp.where(ok, loc, SLABW)
                    pltpu.sync_copy(vals, acc.at[idxb], add=True)
                    return 0
                lax.fori_loop(0, NCHK, do_chunk, 0)
                pltpu.sync_copy(zv.at[pl.ds(0, CH)], acc.at[idxb], add=True)
                pltpu.sync_copy(acc.at[pl.ds(SLABW, L)], drb)
                plsc.subcore_barrier()

            scatter_pass(ones_b)
            pltpu.sync_copy(acc.at[pl.ds(sid * (SLABW // NS), SLABW // NS)],
                            out_h.at[pl.ds(word0 + sid * (SLABW // NS), SLABW // NS)])
            plsc.subcore_barrier()
            scatter_pass(mns_b)
            return 0
        lax.fori_loop(0, NSLAB // NC, do_slab, 0)

    fn = pl.kernel(
        body,
        out_type=jax.ShapeDtypeStruct((NN * NN,), f32),
        mesh=_sc_mesh(),
        compiler_params=pltpu.CompilerParams(use_tc_tiling_on_sc=False),
        scratch_types=[
            pltpu.VMEM_SHARED((SLABW + L,), f32),
            pltpu.VMEM((EPS,), i32), pltpu.VMEM((EPS,), i32),
            pltpu.VMEM((CH,), i32),
            pltpu.VMEM((CH,), f32),
            pltpu.VMEM((CH,), f32),
            pltpu.VMEM((ZW,), f32),
            pltpu.VMEM((L,), f32),
        ],
    )
    return fn(edst, esrc, tok)


def _sc_prop(table2, gat, sct, w, tok):
    _, n, d2 = table2.shape
    EPS = EE // NS
    CH = 128
    NCHK = EPS // CH
    RPS = NN // NS
    weighted = w is not None

    def body(*refs):
        if weighted:
            (tab_h, gat_h, sct_h, w_h, tok_h, out_h,
             acc, gidx, sidx, rows0, rows1, zrows, drb, tkv, wv, sem) = refs
        else:
            (tab_h, gat_h, sct_h, tok_h, out_h,
             acc, gidx, sidx, rows0, rows1, zrows, drb, tkv, sem) = refs
        rows = (rows0, rows1)
        cid = lax.axis_index("c")
        sid = lax.axis_index("s")
        pltpu.sync_copy(tok_h, tkv)
        pltpu.sync_copy(gat_h.at[pl.ds(sid * NCHK, NCHK)], gidx)
        pltpu.sync_copy(sct_h.at[pl.ds(sid * NCHK, NCHK)], sidx)
        if weighted:
            pltpu.sync_copy(w_h.at[pl.ds(sid * EPS, EPS)], wv.at[pl.ds(0, EPS)])
        _zero_vmem_rows(zrows, CH, d2)
        for z in range(RPS // CH):
            pltpu.sync_copy(zrows, acc.at[pl.ds(sid * RPS + z * CH, CH)])
        plsc.subcore_barrier()
        pending = pltpu.async_copy(tab_h.at[cid].at[gidx.at[0]], rows[0], sem)
        for c in range(NCHK):
            pending.wait()
            if c + 1 < NCHK:
                pending = pltpu.async_copy(
                    tab_h.at[cid].at[gidx.at[c + 1]], rows[(c + 1) % 2], sem)
            rb = rows[c % 2]
            if weighted:
                def scale(e, _):
                    ws = wv[pl.ds(c * CH + e, L)][0]
                    for g in range(d2 // L):
                        rb[e, pl.ds(g * L, L)] = rb[e, pl.ds(g * L, L)] * ws
                    return 0
                lax.fori_loop(0, CH, scale, 0)
            pltpu.sync_copy(rb, acc.at[sidx.at[c]], add=True)
        pltpu.sync_copy(zrows, acc.at[sidx.at[NCHK - 1]], add=True)
        pltpu.sync_copy(acc.at[pl.ds(0, 1)], drb)
        plsc.subcore_barrier()
        pltpu.sync_copy(acc.at[pl.ds(sid * RPS, RPS)],
                        out_h.at[cid, pl.ds(sid * RPS, RPS)])

    scratch = [
        pltpu.VMEM_SHARED((NN, d2), f32),
        pltpu.VMEM((NCHK, CH), i32),
        pltpu.VMEM((NCHK, CH), i32),
        pltpu.VMEM((CH, d2), f32),
        pltpu.VMEM((CH, d2), f32),
        pltpu.VMEM((CH, d2), f32),
        pltpu.VMEM((1, d2), f32),
        pltpu.VMEM((L,), f32),
    ]
    if weighted:
        scratch.append(pltpu.VMEM((EPS + L,), f32))
    scratch.append(pltpu.SemaphoreType.DMA)
    fn = pl.kernel(
        body,
        out_type=jax.ShapeDtypeStruct((NC, NN, d2), f32),
        mesh=_sc_mesh(),
        compiler_params=pltpu.CompilerParams(use_tc_tiling_on_sc=False),
        scratch_types=scratch,
    )
    args = (table2, gat.reshape(-1, CH), sct.reshape(-1, CH))
    args += ((w,) if weighted else ()) + (tok,)
    return fn(*args)


def _halves(t):
    d2 = t.shape[1] // 2
    return jnp.stack([t[:, :d2], t[:, d2:]])


def _unhalves(p):
    return jnp.concatenate([p[0], p[1]], axis=1)


def _bases_2d(x, gt):
    g = [gt[i][None, :] for i in range(12)]
    B = [jnp.where((x >= g[i]) & (x < g[i + 1]), 1.0, 0.0).astype(f32)
         for i in range(11)]
    for j in range(1, 4):
        B = [(x - g[i]) / (g[i + j] - g[i]) * B[i]
             + (g[i + j + 1] - x) / (g[i + j + 1] - g[i + 1]) * B[i + 1]
             for i in range(len(B) - 1)]
    return B


def _kan_compute(x, gt, bw, sw_ref):
    silu = x * jax.nn.sigmoid(x)
    acc = lax.dot_general(silu, bw, (((1,), (1,)), ((), ())),
                          preferred_element_type=f32)
    for j, Bj in enumerate(_bases_2d(x, gt)):
        acc += lax.dot_general(Bj, sw_ref[j], (((1,), (1,)), ((), ())),
                               preferred_element_type=f32)
    return acc


def _kan(parts, pre, p, posts, blk=256):
    n, din = parts[0].shape
    dout = p['base_w'].shape[0]
    nparts, npost = len(parts), len(posts)
    has_pre = pre is not None
    gt = jnp.transpose(p['grid'])
    sw8 = jnp.transpose(p['spline_w'], (2, 0, 1))

    def kbody(*refs):
        k = 0
        x = refs[0][...]
        for t in range(1, nparts):
            x = x + refs[t][...]
        k = nparts
        if has_pre:
            x = x * refs[k][...]
            k += 1
        gt_v = refs[k][...]; k += 1
        bw = refs[k][...]; k += 1
        sw_ref = refs[k]; k += 1
        post_refs = refs[k:k + npost]; k += npost
        out_refs = refs[k:]
        h = _kan_compute(x, gt_v, bw, sw_ref)
        if npost == 0:
            out_refs[0][...] = h
        else:
            for t in range(npost):
                out_refs[t][...] = post_refs[t][...] * h

    grid = (n // blk,)
    rowspec = pl.BlockSpec((blk, din), lambda i: (i, 0))
    vecspec = pl.BlockSpec((blk, 1), lambda i: (i, 0))
    in_specs = [rowspec] * nparts
    args = list(parts)
    if has_pre:
        in_specs.append(vecspec); args.append(pre)
    in_specs += [pl.BlockSpec((12, din), lambda i: (0, 0)),
                 pl.BlockSpec((dout, din), lambda i: (0, 0)),
                 pl.BlockSpec((8, dout, din), lambda i: (0, 0, 0))]
    args += [gt, p['base_w'], sw8]
    in_specs += [vecspec] * npost
    args += list(posts)
    nout = max(npost, 1)
    out = pl.pallas_call(
        kbody, grid=grid, in_specs=in_specs,
        out_specs=[pl.BlockSpec((blk, dout), lambda i: (i, 0))] * nout,
        out_shape=[jax.ShapeDtypeStruct((n, dout), f32)] * nout,
    )(*args)
    return out


def _tc_matmul_A2(abf):
    bm = bn = bk = 512
    I, J, K = NN // bm, NN // bn, NN // bk

    def body(l_ref, r_ref, o_ref, acc):
        i, j, k = pl.program_id(0), pl.program_id(1), pl.program_id(2)
        @pl.when(k == 0)
        def _():
            acc[...] = jnp.zeros((bm, bn), f32)
        acc[...] += lax.dot_general(l_ref[...], r_ref[...],
                                    (((1,), (0,)), ((), ())),
                                    preferred_element_type=f32)
        @pl.when(k == K - 1)
        def _():
            ri = lax.broadcasted_iota(i32, (bm, bn), 0) + i * bm
            ci = lax.broadcasted_iota(i32, (bm, bn), 1) + j * bn
            o_ref[...] = jnp.where(ri == ci, 0.0, acc[...])

    return pl.pallas_call(
        body, grid=(I, J, K),
        in_specs=[pl.BlockSpec((bm, bk), lambda i, j, k: (i, k)),
                  pl.BlockSpec((bk, bn), lambda i, j, k: (k, j))],
        out_specs=pl.BlockSpec((bm, bn), lambda i, j, k: (i, j)),
        out_shape=jax.ShapeDtypeStruct((NN, NN), f32),
        scratch_shapes=[pltpu.VMEM((bm, bn), f32)],
        compiler_params=pltpu.CompilerParams(
            dimension_semantics=("parallel", "parallel", "arbitrary")),
    )(abf, abf)


def _tc_degsum(m):
    bm = 512
    I = NN // bm

    def body(m_ref, rs_ref, cs_ref):
        blk = m_ref[...]
        rs_ref[...] = jnp.sum(blk, axis=1, keepdims=True)
        cs_ref[...] = jnp.sum(blk, axis=0, keepdims=True)[None]

    rs, cs = pl.pallas_call(
        body, grid=(I,),
        in_specs=[pl.BlockSpec((bm, NN), lambda i: (i, 0))],
        out_specs=[pl.BlockSpec((bm, 1), lambda i: (i, 0)),
                   pl.BlockSpec((1, 1, NN), lambda i: (i, 0, 0))],
        out_shape=[jax.ShapeDtypeStruct((NN, 1), f32),
                   jax.ShapeDtypeStruct((I, 1, NN), f32)],
    )(m)
    return rs, cs.reshape(I, NN)


def _tc_finalize_sp(degs_t):
    def body(d_ref, o_ref):
        d = d_ref[...]
        def rs_of(col):
            v = d[:, col:col + 1] + d[:, col + 4:col + 5]
            return lax.rsqrt(jnp.maximum(v, 1e-12))
        b = rs_of(0)
        a = rs_of(1)
        bs = rs_of(2)
        a_s = rs_of(3)
        o_ref[...] = jnp.concatenate([a, b, a_s, bs], axis=1)

    return pl.pallas_call(
        body,
        in_specs=[pl.BlockSpec((NN, 8), lambda: (0, 0))],
        out_specs=pl.BlockSpec((NN, 4), lambda: (0, 0)),
        out_shape=jax.ShapeDtypeStruct((NN, 4), f32),
    )(degs_t)


def _tc_finalize_dn(rowsum, colsum_t):
    nI = colsum_t.shape[1]

    def body(r_ref, c_ref, o_ref):
        rd = lax.rsqrt(jnp.maximum(r_ref[...], 1e-12))
        cs = jnp.sum(c_ref[...], axis=1, keepdims=True)
        rs = lax.rsqrt(jnp.maximum(cs, 1e-12))
        o_ref[...] = jnp.concatenate([rd, rs], axis=1)

    return pl.pallas_call(
        body,
        in_specs=[pl.BlockSpec((NN, 1), lambda: (0, 0)),
                  pl.BlockSpec((NN, nI), lambda: (0, 0))],
        out_specs=pl.BlockSpec((NN, 2), lambda: (0, 0)),
        out_shape=jax.ShapeDtypeStruct((NN, 2), f32),
    )(rowsum, colsum_t)


def _mm_prop(m, u, trans):
    n, d = u.shape
    bm, bk = 512, 512
    I, K = NN // bm, NN // bk

    def body(l_ref, r_ref, o_ref, acc):
        k = pl.program_id(1)
        @pl.when(k == 0)
        def _():
            acc[...] = jnp.zeros((bm, d), f32)
        dn = (((0,), (0,)), ((), ())) if trans else (((1,), (0,)), ((), ()))
        acc[...] += lax.dot_general(l_ref[...], r_ref[...], dn,
                                    preferred_element_type=f32)
        @pl.when(k == K - 1)
        def _():
            o_ref[...] = acc[...]

    if trans:
        lspec = pl.BlockSpec((bk, bm), lambda i, k: (k, i))
    else:
        lspec = pl.BlockSpec((bm, bk), lambda i, k: (i, k))
    return pl.pallas_call(
        body, grid=(I, K),
        in_specs=[lspec, pl.BlockSpec((bk, d), lambda i, k: (k, 0))],
        out_specs=pl.BlockSpec((bm, d), lambda i, k: (i, 0)),
        out_shape=jax.ShapeDtypeStruct((NN, d), f32),
        scratch_shapes=[pltpu.VMEM((bm, d), f32)],
        compiler_params=pltpu.CompilerParams(
            dimension_semantics=("parallel", "arbitrary")),
    )(m, u)


def _fu_assemble(fp, feat_cfgs, blk=512):
    d = feat_cfgs[0][0][0].shape[1]
    att = fp['att']
    npart = [len(c[0]) for c in feat_cfgs]

    def body(*refs):
        i = pl.program_id(0)
        k = 0
        feats = []
        for t in range(3):
            x = refs[k][...]
            for _ in range(1, npart[t]):
                k += 1
                x = x + refs[k][...]
            k += 1
            x = x * refs[k][...]
            k += 1
            feats.append(x)
        lng = refs[k][...]; k += 1
        lnb = refs[k][...]; k += 1
        W = refs[k][...]; k += 1
        bv = refs[k][...]; k += 1
        qv = refs[k][...]; k += 1
        F_ref, wf_ref, cs_ref = refs[k], refs[k + 1], refs[k + 2]

        logits = []
        for t in range(3):
            f = feats[t]
            m = jnp.mean(f, axis=1, keepdims=True)
            v = jnp.mean((f - m) ** 2, axis=1, keepdims=True)
            fn = (f - m) * lax.rsqrt(v + 1e-5) * lng + lnb
            tt = jnp.tanh(lax.dot_general(fn, W, (((1,), (1,)), ((), ())),
                                          preferred_element_type=f32) + bv)
            logits.append(lax.dot_general(tt, qv, (((1,), (1,)), ((), ())),
                                          preferred_element_type=f32))
        mx = jnp.maximum(jnp.maximum(logits[0], logits[1]), logits[2])
        es = [jnp.exp(lg - mx) for lg in logits]
        den = es[0] + es[1] + es[2]
        wf = (es[0] * feats[0] + es[1] * feats[1] + es[2] * feats[2]) / den
        F = jnp.concatenate(feats, axis=1)
        F_ref[...] = F
        wf_ref[...] = wf
        @pl.when(i == 0)
        def _():
            cs_ref[...] = jnp.zeros((1, 3 * d), f32)
        cs_ref[...] += jnp.sum(F, axis=0, keepdims=True)

    grid = (NN // blk,)
    rowspec = pl.BlockSpec((blk, d), lambda i: (i, 0))
    vecspec = pl.BlockSpec((blk, 1), lambda i: (i, 0))
    in_specs, args = [], []
    for parts, scale in feat_cfgs:
        in_specs += [rowspec] * len(parts) + [vecspec]
        args += list(parts) + [scale]
    in_specs += [pl.BlockSpec((1, d), lambda i: (0, 0))] * 2
    args += [att['ln_g'][None, :], att['ln_b'][None, :]]
    in_specs += [pl.BlockSpec((d, d), lambda i: (0, 0))]
    args += [att['W']]
    in_specs += [pl.BlockSpec((1, d), lambda i: (0, 0))] * 2
    args += [att['b'][None, :], att['q'][None, :]]
    return pl.pallas_call(
        body, grid=grid, in_specs=in_specs,
        out_specs=[pl.BlockSpec((blk, 3 * d), lambda i: (i, 0)),
                   pl.BlockSpec((blk, d), lambda i: (i, 0)),
                   pl.BlockSpec((1, 3 * d), lambda i: (0, 0))],
        out_shape=[jax.ShapeDtypeStruct((NN, 3 * d), f32),
                   jax.ShapeDtypeStruct((NN, d), f32),
                   jax.ShapeDtypeStruct((1, 3 * d), f32)],
    )(*args)


def _fu_sumsq(F, cs, blk=512):
    dcols = F.shape[1]

    def body(f_ref, c_ref, o_ref):
        i = pl.program_id(0)
        m = c_ref[...] * (1.0 / NN)
        dev = f_ref[...] - m
        @pl.when(i == 0)
        def _():
            o_ref[...] = jnp.zeros((1, dcols), f32)
        o_ref[...] += jnp.sum(dev * dev, axis=0, keepdims=True)

    return pl.pallas_call(
        body, grid=(NN // blk,),
        in_specs=[pl.BlockSpec((blk, dcols), lambda i: (i, 0)),
                  pl.BlockSpec((1, dcols), lambda i: (0, 0))],
        out_specs=pl.BlockSpec((1, dcols), lambda i: (0, 0)),
        out_shape=jax.ShapeDtypeStruct((1, dcols), f32),
    )(F, cs)


def _fu_kan(F, cs, ss, fp, blk=256):
    dcols = F.shape[1]
    p = fp['kan']
    dout = p['base_w'].shape[0]
    gt = jnp.transpose(p['grid'])
    sw8 = jnp.transpose(p['spline_w'], (2, 0, 1))

    def body(f_ref, c_ref, s_ref, g_ref, b_ref, gt_ref, bw_ref, sw_ref,
             o_ref, oc_ref):
        i = pl.program_id(0)
        m = c_ref[...] * (1.0 / NN)
        v = s_ref[...] * (1.0 / NN)
        xb = (f_ref[...] - m) * lax.rsqrt(v + 1e-5) * g_ref[...] + b_ref[...]
        h = _kan_compute(xb, gt_ref[...], bw_ref[...], sw_ref)
        o_ref[...] = h
        @pl.when(i == 0)
        def _():
            oc_ref[...] = jnp.zeros((1, dout), f32)
        oc_ref[...] += jnp.sum(h, axis=0, keepdims=True)

    cspec = pl.BlockSpec((1, dcols), lambda i: (0, 0))
    return pl.pallas_call(
        body, grid=(NN // blk,),
        in_specs=[pl.BlockSpec((blk, dcols), lambda i: (i, 0)), cspec, cspec,
                  cspec, cspec,
                  pl.BlockSpec((12, dcols), lambda i: (0, 0)),
                  pl.BlockSpec((dout, dcols), lambda i: (0, 0)),
                  pl.BlockSpec((8, dout, dcols), lambda i: (0, 0, 0))],
        out_specs=[pl.BlockSpec((blk, dout), lambda i: (i, 0)),
                   pl.BlockSpec((1, dout), lambda i: (0, 0))],
        out_shape=[jax.ShapeDtypeStruct((NN, dout), f32),
                   jax.ShapeDtypeStruct((1, dout), f32)],
    )(F, cs, ss, fp['bn1_g'][None, :], fp['bn1_b'][None, :], gt,
      p['base_w'], sw8)


def _fu_final(wf, kout, kcs, kss, fp, blk=512):
    d = kout.shape[1]

    def body(w_ref, k_ref, c_ref, s_ref, g_ref, b_ref, o_ref):
        m = c_ref[...] * (1.0 / NN)
        v = s_ref[...] * (1.0 / NN)
        h = (k_ref[...] - m) * lax.rsqrt(v + 1e-5) * g_ref[...] + b_ref[...]
        o_ref[...] = jnp.concatenate([w_ref[...], h], axis=1)

    cspec = pl.BlockSpec((1, d), lambda i: (0, 0))
    return pl.pallas_call(
        body, grid=(NN // blk,),
        in_specs=[pl.BlockSpec((blk, d), lambda i: (i, 0)),
                  pl.BlockSpec((blk, d), lambda i: (i, 0)),
                  cspec, cspec, cspec, cspec],
        out_specs=pl.BlockSpec((blk, 2 * d), lambda i: (i, 0)),
        out_shape=jax.ShapeDtypeStruct((NN, 2 * d), f32),
    )(wf, kout, kcs, kss, fp['bn2_g'][None, :], fp['bn2_b'][None, :])


def _fusion(fp, f1, f2, f3):
    F, wf, cs = _fu_assemble(fp, [f1, f2, f3])
    ss = _fu_sumsq(F, cs)
    kout, kcs = _fu_kan(F, cs, ss, fp)
    kss = _fu_sumsq(kout, kcs)
    return _fu_final(wf, kout, kcs, kss, fp)


def kernel(x, edge_index, sim_index, sim_weight, params):
    src, dst = edge_index[0], edge_index[1]
    ssrc, sdst = sim_index[0], sim_index[1]

    degs = _sc_degrees(dst, src, sdst, ssrc, sim_weight)
    degs_t = jnp.transpose(degs.reshape(8, NN))
    Aflat = _sc_build_adj(dst, src, degs[0, :L])
    A = Aflat.reshape(NN, NN)
    A2 = _tc_matmul_A2(A.astype(jnp.bfloat16))
    rowsum, colsum_part = _tc_degsum(A2)
    scales = _tc_finalize_sp(degs_t)
    a_ = scales[:, 0:1]
    b_ = scales[:, 1:2]
    as_ = scales[:, 2:3]
    bs_ = scales[:, 3:4]
    scales2 = _tc_finalize_dn(rowsum, jnp.transpose(colsum_part))
    rd = scales2[:, 0:1]
    rs = scales2[:, 1:2]

    pdd, pco, psm = params['ddi'], params['co'], params['sim']
    t_in, t_out = _kan([x], None, pdd['kan1'], [a_, b_])
    t_sim = _kan([x], None, psm['kan1'], [as_])[0]
    u_in, u_out = _kan([x], None, pco['kan1'], [rs, rd])

    P_in = _sc_prop(_halves(t_in), src, dst, None, Aflat[:L])
    P_out = _sc_prop(_halves(t_out), dst, src, None, P_in[0, 0, :L])
    S1 = _sc_prop(_halves(t_sim), ssrc, sdst, sim_weight, P_out[0, 0, :L])
    Y_in = _mm_prop(A2, u_in, False)
    Y_out = _mm_prop(A2, u_out, True)

    t2_in, = _kan([_unhalves(P_in)], b_, pdd['kan2'], [a_])
    t2_out, = _kan([_unhalves(P_out)], a_, pdd['kan2'], [b_])
    t2_sim, = _kan([_unhalves(S1)], bs_, psm['kan2'], [as_])
    u2_in, = _kan([Y_in], rd, pco['kan2'], [rs])
    u2_out, = _kan([Y_out], rs, pco['kan2'], [rd])

    X_in = _sc_prop(_halves(t2_in), src, dst, None, S1[0, 0, :L])
    X_out = _sc_prop(_halves(t2_out), dst, src, None, X_in[0, 0, :L])
    S2 = _sc_prop(_halves(t2_sim), ssrc, sdst, sim_weight, X_out[0, 0, :L])
    Y2_in = _mm_prop(A2, u2_in, False)
    Y2_out = _mm_prop(A2, u2_out, True)

    x_sim_f = _unhalves(S2)
    x_in = _fusion(params['in_fusion'],
                   ([_unhalves(X_in)], b_), ([Y2_in], rd), ([x_sim_f], bs_))
    x_out = _fusion(params['out_fusion'],
                    ([_unhalves(X_out)], a_), ([Y2_out], rs), ([x_sim_f], bs_))
    return (x_in, x_out)


# speedup vs baseline: 1.2629x; 1.2629x over previous
import functools

import jax
import jax.numpy as jnp
from jax import lax
from jax.experimental import pallas as pl
from jax.experimental.pallas import tpu as pltpu
from jax.experimental.pallas import tpu_sc as plsc

NN = 4096
EE = 65536
NC, NS, L = 2, 16, 16

f32 = jnp.float32
i32 = jnp.int32


def _sc_mesh():
    return plsc.VectorSubcoreMesh(core_axis_name="c", subcore_axis_name="s")


def _zero_vmem_rows(ref, nrows, width):
    def body(e, _):
        for g in range(width // L):
            ref[e, pl.ds(g * L, L)] = jnp.zeros((L,), f32)
        return 0
    lax.fori_loop(0, nrows, body, 0)


def _zero_vmem_1d(ref, n):
    def body(e, _):
        ref[pl.ds(e * L, L)] = jnp.zeros((L,), f32)
        return 0
    lax.fori_loop(0, n // L, body, 0)


def _sc_degrees(edst, esrc, sdst, ssrc, w):
    EPS = EE // (NC * NS)
    CH = 128
    NCHK = EPS // CH
    AW = 4 * NN

    def body(edst_h, esrc_h, sdst_h, ssrc_h, w_h, out_h,
             acc, dstv, srcv, sdv, ssv, wv, idxb, ones_b, zv, drb):
        cid = lax.axis_index("c")
        sid = lax.axis_index("s")
        base = (cid * NS + sid) * EPS
        pltpu.sync_copy(edst_h.at[pl.ds(base, EPS)], dstv)
        pltpu.sync_copy(esrc_h.at[pl.ds(base, EPS)], srcv)
        pltpu.sync_copy(sdst_h.at[pl.ds(base, EPS)], sdv)
        pltpu.sync_copy(ssrc_h.at[pl.ds(base, EPS)], ssv)
        pltpu.sync_copy(w_h.at[pl.ds(base, EPS)], wv)
        _zero_vmem_1d(zv, 1024)

        def fill_ones(e, _):
            ones_b[pl.ds(e * L, L)] = jnp.full((L,), 1.0, f32)
            return 0
        lax.fori_loop(0, CH // L, fill_ones, 0)

        pltpu.sync_copy(zv, acc.at[pl.ds(sid * 1024, 1024)])
        plsc.subcore_barrier()

        for c in range(NCHK):
            for (vec, off, vals) in (
                    (dstv, 0, ones_b), (srcv, NN, ones_b),
                    (sdv, 2 * NN, wv.at[pl.ds(c * CH, CH)]),
                    (ssv, 3 * NN, wv.at[pl.ds(c * CH, CH)])):
                for g in range(CH // L):
                    v = vec[pl.ds(c * CH + g * L, L)]
                    idxb[pl.ds(g * L, L)] = v + off
                pltpu.sync_copy(vals, acc.at[idxb], add=True)
        pltpu.sync_copy(zv.at[pl.ds(0, CH)], acc.at[idxb], add=True)
        pltpu.sync_copy(acc.at[pl.ds(0, L)], drb)
        plsc.subcore_barrier()
        pltpu.sync_copy(acc.at[pl.ds(sid * 1024, 1024)],
                        out_h.at[cid, pl.ds(sid * 1024, 1024)])

    fn = pl.kernel(
        body,
        out_type=jax.ShapeDtypeStruct((NC, AW), f32),
        mesh=_sc_mesh(),
        compiler_params=pltpu.CompilerParams(use_tc_tiling_on_sc=False),
        scratch_types=[
            pltpu.VMEM_SHARED((AW,), f32),
            pltpu.VMEM((EPS,), i32), pltpu.VMEM((EPS,), i32),
            pltpu.VMEM((EPS,), i32), pltpu.VMEM((EPS,), i32),
            pltpu.VMEM((EPS,), f32),
            pltpu.VMEM((CH,), i32),
            pltpu.VMEM((CH,), f32),
            pltpu.VMEM((1024,), f32),
            pltpu.VMEM((L,), f32),
        ],
    )
    return fn(edst, esrc, sdst, ssrc, w)


def _sc_build_adj(edst, esrc, tok):
    EPS = EE // NS
    CH = 128
    NCHK = EPS // CH
    SLABW = 256 * NN
    NSLAB = (NN * NN) // SLABW
    ZW = 16384

    def body(edst_h, esrc_h, tok_h, out_h,
             acc, dstv, srcv, idxb, ones_b, zv, drb):
        cid = lax.axis_index("c")
        sid = lax.axis_index("s")
        base = sid * EPS
        pltpu.sync_copy(tok_h, drb)
        pltpu.sync_copy(edst_h.at[pl.ds(base, EPS)], dstv)
        pltpu.sync_copy(esrc_h.at[pl.ds(base, EPS)], srcv)
        _zero_vmem_1d(zv, ZW)

        def fill_ones(e, _):
            ones_b[pl.ds(e * L, L)] = jnp.full((L,), 1.0, f32)
            return 0
        lax.fori_loop(0, CH // L, fill_ones, 0)

        def do_slab(t, _):
            slab = 2 * t + cid
            word0 = slab * SLABW
            for z in range(SLABW // NS // ZW):
                pltpu.sync_copy(zv, acc.at[pl.ds(sid * (SLABW // NS) + z * ZW, ZW)])
            @pl.when(sid == 0)
            def _():
                pltpu.sync_copy(zv.at[pl.ds(0, L)], acc.at[pl.ds(SLABW, L)])
            plsc.subcore_barrier()

            def do_chunk(c, _):
                for g in range(CH // L):
                    d = dstv[pl.ds(c * CH + g * L, L)]
                    s = srcv[pl.ds(c * CH + g * L, L)]
                    loc = d * NN + s - word0
                    ok = (loc >= 0) & (loc < SLABW)
                    idxb[pl.ds(g * L, L)] = jnp.where(ok, loc, SLABW)
                pltpu.sync_copy(ones_b, acc.at[idxb], add=True)
                return 0
            lax.fori_loop(0, NCHK, do_chunk, 0)
            pltpu.sync_copy(zv.at[pl.ds(0, CH)], acc.at[idxb], add=True)
            pltpu.sync_copy(acc.at[pl.ds(SLABW, L)], drb)
            plsc.subcore_barrier()
            pltpu.sync_copy(acc.at[pl.ds(sid * (SLABW // NS), SLABW // NS)],
                            out_h.at[pl.ds(word0 + sid * (SLABW // NS), SLABW // NS)])
            plsc.subcore_barrier()
            return 0
        lax.fori_loop(0, NSLAB // NC, do_slab, 0)

    fn = pl.kernel(
        body,
        out_type=jax.ShapeDtypeStruct((NN * NN,), f32),
        mesh=_sc_mesh(),
        compiler_params=pltpu.CompilerParams(use_tc_tiling_on_sc=False),
        scratch_types=[
            pltpu.VMEM_SHARED((SLABW + L,), f32),
            pltpu.VMEM((EPS,), i32), pltpu.VMEM((EPS,), i32),
            pltpu.VMEM((CH,), i32),
            pltpu.VMEM((CH,), f32),
            pltpu.VMEM((ZW,), f32),
            pltpu.VMEM((L,), f32),
        ],
    )
    return fn(edst, esrc, tok)


def _sc_prop(table2, gat, sct, w, tok):
    _, n, d2 = table2.shape
    EPS = EE // NS
    CH = 128
    NCHK = EPS // CH
    RPS = NN // NS
    weighted = w is not None

    def body(*refs):
        if weighted:
            (tab_h, gat_h, sct_h, w_h, tok_h, out_h,
             acc, gidx, sidx, rows0, rows1, zrows, drb, tkv, wv, sem) = refs
        else:
            (tab_h, gat_h, sct_h, tok_h, out_h,
             acc, gidx, sidx, rows0, rows1, zrows, drb, tkv, sem) = refs
        rows = (rows0, rows1)
        cid = lax.axis_index("c")
        sid = lax.axis_index("s")
        pltpu.sync_copy(tok_h, tkv)
        pltpu.sync_copy(gat_h.at[pl.ds(sid * NCHK, NCHK)], gidx)
        pltpu.sync_copy(sct_h.at[pl.ds(sid * NCHK, NCHK)], sidx)
        if weighted:
            pltpu.sync_copy(w_h.at[pl.ds(sid * EPS, EPS)], wv.at[pl.ds(0, EPS)])
        _zero_vmem_rows(zrows, CH, d2)
        for z in range(RPS // CH):
            pltpu.sync_copy(zrows, acc.at[pl.ds(sid * RPS + z * CH, CH)])
        plsc.subcore_barrier()
        pending = pltpu.async_copy(tab_h.at[cid].at[gidx.at[0]], rows[0], sem)
        for c in range(NCHK):
            pending.wait()
            if c + 1 < NCHK:
                pending = pltpu.async_copy(
                    tab_h.at[cid].at[gidx.at[c + 1]], rows[(c + 1) % 2], sem)
            rb = rows[c % 2]
            if weighted:
                def scale(e, _):
                    ws = wv[pl.ds(c * CH + e, L)][0]
                    for g in range(d2 // L):
                        rb[e, pl.ds(g * L, L)] = rb[e, pl.ds(g * L, L)] * ws
                    return 0
                lax.fori_loop(0, CH, scale, 0)
            pltpu.sync_copy(rb, acc.at[sidx.at[c]], add=True)
        pltpu.sync_copy(zrows, acc.at[sidx.at[NCHK - 1]], add=True)
        pltpu.sync_copy(acc.at[pl.ds(0, 1)], drb)
        plsc.subcore_barrier()
        pltpu.sync_copy(acc.at[pl.ds(sid * RPS, RPS)],
                        out_h.at[cid, pl.ds(sid * RPS, RPS)])

    scratch = [
        pltpu.VMEM_SHARED((NN, d2), f32),
        pltpu.VMEM((NCHK, CH), i32),
        pltpu.VMEM((NCHK, CH), i32),
        pltpu.VMEM((CH, d2), f32),
        pltpu.VMEM((CH, d2), f32),
        pltpu.VMEM((CH, d2), f32),
        pltpu.VMEM((1, d2), f32),
        pltpu.VMEM((L,), f32),
    ]
    if weighted:
        scratch.append(pltpu.VMEM((EPS + L,), f32))
    scratch.append(pltpu.SemaphoreType.DMA)
    fn = pl.kernel(
        body,
        out_type=jax.ShapeDtypeStruct((NC, NN, d2), f32),
        mesh=_sc_mesh(),
        compiler_params=pltpu.CompilerParams(use_tc_tiling_on_sc=False),
        scratch_types=scratch,
    )
    args = (table2, gat.reshape(-1, CH), sct.reshape(-1, CH))
    args += ((w,) if weighted else ()) + (tok,)
    return fn(*args)


def _halves(t):
    d2 = t.shape[1] // 2
    return jnp.stack([t[:, :d2], t[:, d2:]])


def _unhalves(p):
    return jnp.concatenate([p[0], p[1]], axis=1)


def _bases_2d(x, gt):
    g = [gt[i][None, :] for i in range(12)]
    B = [jnp.where((x >= g[i]) & (x < g[i + 1]), 1.0, 0.0).astype(f32)
         for i in range(11)]
    for j in range(1, 4):
        B = [(x - g[i]) / (g[i + j] - g[i]) * B[i]
             + (g[i + j + 1] - x) / (g[i + j + 1] - g[i + 1]) * B[i + 1]
             for i in range(len(B) - 1)]
    return B


def _kan_compute(x, gt, bw, sw_ref):
    silu = x * jax.nn.sigmoid(x)
    acc = lax.dot_general(silu, bw, (((1,), (1,)), ((), ())),
                          preferred_element_type=f32)
    for j, Bj in enumerate(_bases_2d(x, gt)):
        acc += lax.dot_general(Bj, sw_ref[j], (((1,), (1,)), ((), ())),
                               preferred_element_type=f32)
    return acc


def _kan(parts, pre, p, posts, blk=256):
    n, din = parts[0].shape
    dout = p['base_w'].shape[0]
    nparts, npost = len(parts), len(posts)
    has_pre = pre is not None
    gt = jnp.transpose(p['grid'])
    sw8 = jnp.transpose(p['spline_w'], (2, 0, 1))

    def kbody(*refs):
        k = 0
        x = refs[0][...]
        for t in range(1, nparts):
            x = x + refs[t][...]
        k = nparts
        if has_pre:
            x = x * refs[k][...]
            k += 1
        gt_v = refs[k][...]; k += 1
        bw = refs[k][...]; k += 1
        sw_ref = refs[k]; k += 1
        post_refs = refs[k:k + npost]; k += npost
        out_refs = refs[k:]
        h = _kan_compute(x, gt_v, bw, sw_ref)
        if npost == 0:
            out_refs[0][...] = h
        else:
            for t in range(npost):
                out_refs[t][...] = post_refs[t][...] * h

    grid = (n // blk,)
    rowspec = pl.BlockSpec((blk, din), lambda i: (i, 0))
    vecspec = pl.BlockSpec((blk, 1), lambda i: (i, 0))
    in_specs = [rowspec] * nparts
    args = list(parts)
    if has_pre:
        in_specs.append(vecspec); args.append(pre)
    in_specs += [pl.BlockSpec((12, din), lambda i: (0, 0)),
                 pl.BlockSpec((dout, din), lambda i: (0, 0)),
                 pl.BlockSpec((8, dout, din), lambda i: (0, 0, 0))]
    args += [gt, p['base_w'], sw8]
    in_specs += [vecspec] * npost
    args += list(posts)
    nout = max(npost, 1)
    out = pl.pallas_call(
        kbody, grid=grid, in_specs=in_specs,
        out_specs=[pl.BlockSpec((blk, dout), lambda i: (i, 0))] * nout,
        out_shape=[jax.ShapeDtypeStruct((n, dout), f32)] * nout,
    )(*args)
    return out


def _tc_matmul_A2(abf):
    bm = bn = bk = 512
    I, J, K = NN // bm, NN // bn, NN // bk

    def body(l_ref, r_ref, o_ref, acc):
        i, j, k = pl.program_id(0), pl.program_id(1), pl.program_id(2)
        @pl.when(k == 0)
        def _():
            acc[...] = jnp.zeros((bm, bn), f32)
        acc[...] += lax.dot_general(l_ref[...], r_ref[...],
                                    (((1,), (0,)), ((), ())),
                                    preferred_element_type=f32)
        @pl.when(k == K - 1)
        def _():
            ri = lax.broadcasted_iota(i32, (bm, bn), 0) + i * bm
            ci = lax.broadcasted_iota(i32, (bm, bn), 1) + j * bn
            o_ref[...] = jnp.where(ri == ci, 0.0, acc[...])

    return pl.pallas_call(
        body, grid=(I, J, K),
        in_specs=[pl.BlockSpec((bm, bk), lambda i, j, k: (i, k)),
                  pl.BlockSpec((bk, bn), lambda i, j, k: (k, j))],
        out_specs=pl.BlockSpec((bm, bn), lambda i, j, k: (i, j)),
        out_shape=jax.ShapeDtypeStruct((NN, NN), f32),
        scratch_shapes=[pltpu.VMEM((bm, bn), f32)],
        compiler_params=pltpu.CompilerParams(
            dimension_semantics=("parallel", "parallel", "arbitrary")),
    )(abf, abf)


def _tc_degsum(m):
    bm = 512
    I = NN // bm

    def body(m_ref, rs_ref, cs_ref):
        blk = m_ref[...]
        rs_ref[...] = jnp.sum(blk, axis=1, keepdims=True)
        cs_ref[...] = jnp.sum(blk, axis=0, keepdims=True)[None]

    rs, cs = pl.pallas_call(
        body, grid=(I,),
        in_specs=[pl.BlockSpec((bm, NN), lambda i: (i, 0))],
        out_specs=[pl.BlockSpec((bm, 1), lambda i: (i, 0)),
                   pl.BlockSpec((1, 1, NN), lambda i: (i, 0, 0))],
        out_shape=[jax.ShapeDtypeStruct((NN, 1), f32),
                   jax.ShapeDtypeStruct((I, 1, NN), f32)],
    )(m)
    return rs, cs.reshape(I, NN)


def _tc_finalize_sp(degs_t):
    def body(d_ref, o_ref):
        d = d_ref[...]
        def rs_of(col):
            v = d[:, col:col + 1] + d[:, col + 4:col + 5]
            return lax.rsqrt(jnp.maximum(v, 1e-12))
        b = rs_of(0)
        a = rs_of(1)
        bs = rs_of(2)
        a_s = rs_of(3)
        o_ref[...] = jnp.concatenate([a, b, a_s, bs], axis=1)

    return pl.pallas_call(
        body,
        in_specs=[pl.BlockSpec((NN, 8), lambda: (0, 0))],
        out_specs=pl.BlockSpec((NN, 4), lambda: (0, 0)),
        out_shape=jax.ShapeDtypeStruct((NN, 4), f32),
    )(degs_t)


def _tc_finalize_dn(rowsum, colsum_t):
    nI = colsum_t.shape[1]

    def body(r_ref, c_ref, o_ref):
        rd = lax.rsqrt(jnp.maximum(r_ref[...], 1e-12))
        cs = jnp.sum(c_ref[...], axis=1, keepdims=True)
        rs = lax.rsqrt(jnp.maximum(cs, 1e-12))
        o_ref[...] = jnp.concatenate([rd, rs], axis=1)

    return pl.pallas_call(
        body,
        in_specs=[pl.BlockSpec((NN, 1), lambda: (0, 0)),
                  pl.BlockSpec((NN, nI), lambda: (0, 0))],
        out_specs=pl.BlockSpec((NN, 2), lambda: (0, 0)),
        out_shape=jax.ShapeDtypeStruct((NN, 2), f32),
    )(rowsum, colsum_t)


def _mm_prop(m, u, trans):
    n, d = u.shape
    bm, bk = 512, 512
    I, K = NN // bm, NN // bk

    def body(l_ref, r_ref, o_ref, acc):
        k = pl.program_id(1)
        @pl.when(k == 0)
        def _():
            acc[...] = jnp.zeros((bm, d), f32)
        dn = (((0,), (0,)), ((), ())) if trans else (((1,), (0,)), ((), ()))
        acc[...] += lax.dot_general(l_ref[...], r_ref[...], dn,
                                    preferred_element_type=f32)
        @pl.when(k == K - 1)
        def _():
            o_ref[...] = acc[...]

    if trans:
        lspec = pl.BlockSpec((bk, bm), lambda i, k: (k, i))
    else:
        lspec = pl.BlockSpec((bm, bk), lambda i, k: (i, k))
    return pl.pallas_call(
        body, grid=(I, K),
        in_specs=[lspec, pl.BlockSpec((bk, d), lambda i, k: (k, 0))],
        out_specs=pl.BlockSpec((bm, d), lambda i, k: (i, 0)),
        out_shape=jax.ShapeDtypeStruct((NN, d), f32),
        scratch_shapes=[pltpu.VMEM((bm, d), f32)],
        compiler_params=pltpu.CompilerParams(
            dimension_semantics=("parallel", "arbitrary")),
    )(m, u)


def _fu_assemble(fp, feat_cfgs, blk=512):
    d = feat_cfgs[0][0][0].shape[1]
    att = fp['att']
    npart = [len(c[0]) for c in feat_cfgs]

    def body(*refs):
        i = pl.program_id(0)
        k = 0
        feats = []
        for t in range(3):
            x = refs[k][...]
            for _ in range(1, npart[t]):
                k += 1
                x = x + refs[k][...]
            k += 1
            x = x * refs[k][...]
            k += 1
            feats.append(x)
        lng = refs[k][...]; k += 1
        lnb = refs[k][...]; k += 1
        W = refs[k][...]; k += 1
        bv = refs[k][...]; k += 1
        qv = refs[k][...]; k += 1
        F_ref, wf_ref, cs_ref = refs[k], refs[k + 1], refs[k + 2]

        logits = []
        for t in range(3):
            f = feats[t]
            m = jnp.mean(f, axis=1, keepdims=True)
            v = jnp.mean((f - m) ** 2, axis=1, keepdims=True)
            fn = (f - m) * lax.rsqrt(v + 1e-5) * lng + lnb
            tt = jnp.tanh(lax.dot_general(fn, W, (((1,), (1,)), ((), ())),
                                          preferred_element_type=f32) + bv)
            logits.append(lax.dot_general(tt, qv, (((1,), (1,)), ((), ())),
                                          preferred_element_type=f32))
        mx = jnp.maximum(jnp.maximum(logits[0], logits[1]), logits[2])
        es = [jnp.exp(lg - mx) for lg in logits]
        den = es[0] + es[1] + es[2]
        wf = (es[0] * feats[0] + es[1] * feats[1] + es[2] * feats[2]) / den
        F = jnp.concatenate(feats, axis=1)
        F_ref[...] = F
        wf_ref[...] = wf
        @pl.when(i == 0)
        def _():
            cs_ref[...] = jnp.zeros((1, 3 * d), f32)
        cs_ref[...] += jnp.sum(F, axis=0, keepdims=True)

    grid = (NN // blk,)
    rowspec = pl.BlockSpec((blk, d), lambda i: (i, 0))
    vecspec = pl.BlockSpec((blk, 1), lambda i: (i, 0))
    in_specs, args = [], []
    for parts, scale in feat_cfgs:
        in_specs += [rowspec] * len(parts) + [vecspec]
        args += list(parts) + [scale]
    in_specs += [pl.BlockSpec((1, d), lambda i: (0, 0))] * 2
    args += [att['ln_g'][None, :], att['ln_b'][None, :]]
    in_specs += [pl.BlockSpec((d, d), lambda i: (0, 0))]
    args += [att['W']]
    in_specs += [pl.BlockSpec((1, d), lambda i: (0, 0))] * 2
    args += [att['b'][None, :], att['q'][None, :]]
    return pl.pallas_call(
        body, grid=grid, in_specs=in_specs,
        out_specs=[pl.BlockSpec((blk, 3 * d), lambda i: (i, 0)),
                   pl.BlockSpec((blk, d), lambda i: (i, 0)),
                   pl.BlockSpec((1, 3 * d), lambda i: (0, 0))],
        out_shape=[jax.ShapeDtypeStruct((NN, 3 * d), f32),
                   jax.ShapeDtypeStruct((NN, d), f32),
                   jax.ShapeDtypeStruct((1, 3 * d), f32)],
    )(*args)


def _fu_sumsq(F, cs, blk=512):
    dcols = F.shape[1]

    def body(f_ref, c_ref, o_ref):
        i = pl.program_id(0)
        m = c_ref[...] * (1.0 / NN)
        dev = f_ref[...] - m
        @pl.when(i == 0)
        def _():
            o_ref[...] = jnp.zeros((1, dcols), f32)
        o_ref[...] += jnp.sum(dev * dev, axis=0, keepdims=True)

    return pl.pallas_call(
        body, grid=(NN // blk,),
        in_specs=[pl.BlockSpec((blk, dcols), lambda i: (i, 0)),
                  pl.BlockSpec((1, dcols), lambda i: (0, 0))],
        out_specs=pl.BlockSpec((1, dcols), lambda i: (0, 0)),
        out_shape=jax.ShapeDtypeStruct((1, dcols), f32),
    )(F, cs)


def _fu_kan(F, cs, ss, fp, blk=256):
    dcols = F.shape[1]
    p = fp['kan']
    dout = p['base_w'].shape[0]
    gt = jnp.transpose(p['grid'])
    sw8 = jnp.transpose(p['spline_w'], (2, 0, 1))

    def body(f_ref, c_ref, s_ref, g_ref, b_ref, gt_ref, bw_ref, sw_ref,
             o_ref, oc_ref):
        i = pl.program_id(0)
        m = c_ref[...] * (1.0 / NN)
        v = s_ref[...] * (1.0 / NN)
        xb = (f_ref[...] - m) * lax.rsqrt(v + 1e-5) * g_ref[...] + b_ref[...]
        h = _kan_compute(xb, gt_ref[...], bw_ref[...], sw_ref)
        o_ref[...] = h
        @pl.when(i == 0)
        def _():
            oc_ref[...] = jnp.zeros((1, dout), f32)
        oc_ref[...] += jnp.sum(h, axis=0, keepdims=True)

    cspec = pl.BlockSpec((1, dcols), lambda i: (0, 0))
    return pl.pallas_call(
        body, grid=(NN // blk,),
        in_specs=[pl.BlockSpec((blk, dcols), lambda i: (i, 0)), cspec, cspec,
                  cspec, cspec,
                  pl.BlockSpec((12, dcols), lambda i: (0, 0)),
                  pl.BlockSpec((dout, dcols), lambda i: (0, 0)),
                  pl.BlockSpec((8, dout, dcols), lambda i: (0, 0, 0))],
        out_specs=[pl.BlockSpec((blk, dout), lambda i: (i, 0)),
                   pl.BlockSpec((1, dout), lambda i: (0, 0))],
        out_shape=[jax.ShapeDtypeStruct((NN, dout), f32),
                   jax.ShapeDtypeStruct((1, dout), f32)],
    )(F, cs, ss, fp['bn1_g'][None, :], fp['bn1_b'][None, :], gt,
      p['base_w'], sw8)


def _fu_final(wf, kout, kcs, kss, fp, blk=512):
    d = kout.shape[1]

    def body(w_ref, k_ref, c_ref, s_ref, g_ref, b_ref, o_ref):
        m = c_ref[...] * (1.0 / NN)
        v = s_ref[...] * (1.0 / NN)
        h = (k_ref[...] - m) * lax.rsqrt(v + 1e-5) * g_ref[...] + b_ref[...]
        o_ref[...] = jnp.concatenate([w_ref[...], h], axis=1)

    cspec = pl.BlockSpec((1, d), lambda i: (0, 0))
    return pl.pallas_call(
        body, grid=(NN // blk,),
        in_specs=[pl.BlockSpec((blk, d), lambda i: (i, 0)),
                  pl.BlockSpec((blk, d), lambda i: (i, 0)),
                  cspec, cspec, cspec, cspec],
        out_specs=pl.BlockSpec((blk, 2 * d), lambda i: (i, 0)),
        out_shape=jax.ShapeDtypeStruct((NN, 2 * d), f32),
    )(wf, kout, kcs, kss, fp['bn2_g'][None, :], fp['bn2_b'][None, :])


def _fusion(fp, f1, f2, f3):
    F, wf, cs = _fu_assemble(fp, [f1, f2, f3])
    ss = _fu_sumsq(F, cs)
    kout, kcs = _fu_kan(F, cs, ss, fp)
    kss = _fu_sumsq(kout, kcs)
    return _fu_final(wf, kout, kcs, kss, fp)


def kernel(x, edge_index, sim_index, sim_weight, params):
    src, dst = edge_index[0], edge_index[1]
    ssrc, sdst = sim_index[0], sim_index[1]

    degs = _sc_degrees(dst, src, sdst, ssrc, sim_weight)
    degs_t = jnp.transpose(degs.reshape(8, NN))
    Aflat = _sc_build_adj(dst, src, degs[0, :L])
    A = Aflat.reshape(NN, NN)
    A2 = _tc_matmul_A2(A.astype(jnp.bfloat16))
    rowsum, colsum_part = _tc_degsum(A2)
    scales = _tc_finalize_sp(degs_t)
    a_ = scales[:, 0:1]
    b_ = scales[:, 1:2]
    as_ = scales[:, 2:3]
    bs_ = scales[:, 3:4]
    scales2 = _tc_finalize_dn(rowsum, jnp.transpose(colsum_part))
    rd = scales2[:, 0:1]
    rs = scales2[:, 1:2]

    pdd, pco, psm = params['ddi'], params['co'], params['sim']
    t_in, t_out = _kan([x], None, pdd['kan1'], [a_, b_])
    t_sim = _kan([x], None, psm['kan1'], [as_])[0]
    u_in, u_out = _kan([x], None, pco['kan1'], [rs, rd])

    P_in = _sc_prop(_halves(t_in), src, dst, None, Aflat[:L])
    P_out = _sc_prop(_halves(t_out), dst, src, None, P_in[0, 0, :L])
    S1 = _sc_prop(_halves(t_sim), ssrc, sdst, sim_weight, P_out[0, 0, :L])
    Y_in = _mm_prop(A2, u_in, False)
    Y_out = _mm_prop(A2, u_out, True)

    t2_in, = _kan([_unhalves(P_in)], b_, pdd['kan2'], [a_])
    t2_out, = _kan([_unhalves(P_out)], a_, pdd['kan2'], [b_])
    t2_sim, = _kan([_unhalves(S1)], bs_, psm['kan2'], [as_])
    u2_in, = _kan([Y_in], rd, pco['kan2'], [rs])
    u2_out, = _kan([Y_out], rs, pco['kan2'], [rd])

    X_in = _sc_prop(_halves(t2_in), src, dst, None, S1[0, 0, :L])
    X_out = _sc_prop(_halves(t2_out), dst, src, None, X_in[0, 0, :L])
    S2 = _sc_prop(_halves(t2_sim), ssrc, sdst, sim_weight, X_out[0, 0, :L])
    Y2_in = _mm_prop(A2, u2_in, False)
    Y2_out = _mm_prop(A2, u2_out, True)

    x_sim_f = _unhalves(S2)
    x_in = _fusion(params['in_fusion'],
                   ([_unhalves(X_in)], b_), ([Y2_in], rd), ([x_sim_f], bs_))
    x_out = _fusion(params['out_fusion'],
                    ([_unhalves(X_out)], a_), ([Y2_out], rs), ([x_sim_f], bs_))
    return (x_in, x_out)


# speedup vs baseline: 1.2889x; 1.0206x over previous
import functools

import jax
import jax.numpy as jnp
from jax import lax
from jax.experimental import pallas as pl
from jax.experimental.pallas import tpu as pltpu
from jax.experimental.pallas import tpu_sc as plsc

NN = 4096
EE = 65536
NC, NS, L = 2, 16, 16

f32 = jnp.float32
i32 = jnp.int32


def _sc_mesh():
    return plsc.VectorSubcoreMesh(core_axis_name="c", subcore_axis_name="s")


def _zero_vmem_rows(ref, nrows, width):
    def body(e, _):
        for g in range(width // L):
            ref[e, pl.ds(g * L, L)] = jnp.zeros((L,), f32)
        return 0
    lax.fori_loop(0, nrows, body, 0)


def _zero_vmem_1d(ref, n):
    def body(e, _):
        ref[pl.ds(e * L, L)] = jnp.zeros((L,), f32)
        return 0
    lax.fori_loop(0, n // L, body, 0)


def _sc_degrees(edst, esrc, sdst, ssrc, w):
    EPS = EE // (NC * NS)
    CH = 128
    NCHK = EPS // CH
    AW = 4 * NN

    def body(edst_h, esrc_h, sdst_h, ssrc_h, w_h, out_h,
             acc, dstv, srcv, sdv, ssv, wv, idxb, ones_b, zv, drb):
        cid = lax.axis_index("c")
        sid = lax.axis_index("s")
        base = (cid * NS + sid) * EPS
        pltpu.sync_copy(edst_h.at[pl.ds(base, EPS)], dstv)
        pltpu.sync_copy(esrc_h.at[pl.ds(base, EPS)], srcv)
        pltpu.sync_copy(sdst_h.at[pl.ds(base, EPS)], sdv)
        pltpu.sync_copy(ssrc_h.at[pl.ds(base, EPS)], ssv)
        pltpu.sync_copy(w_h.at[pl.ds(base, EPS)], wv)
        _zero_vmem_1d(zv, 1024)

        def fill_ones(e, _):
            ones_b[pl.ds(e * L, L)] = jnp.full((L,), 1.0, f32)
            return 0
        lax.fori_loop(0, CH // L, fill_ones, 0)

        pltpu.sync_copy(zv, acc.at[pl.ds(sid * 1024, 1024)])
        plsc.subcore_barrier()

        for c in range(NCHK):
            for (vec, off, vals) in (
                    (dstv, 0, ones_b), (srcv, NN, ones_b),
                    (sdv, 2 * NN, wv.at[pl.ds(c * CH, CH)]),
                    (ssv, 3 * NN, wv.at[pl.ds(c * CH, CH)])):
                for g in range(CH // L):
                    v = vec[pl.ds(c * CH + g * L, L)]
                    idxb[pl.ds(g * L, L)] = v + off
                pltpu.sync_copy(vals, acc.at[idxb], add=True)
        pltpu.sync_copy(zv.at[pl.ds(0, CH)], acc.at[idxb], add=True)
        pltpu.sync_copy(acc.at[pl.ds(0, L)], drb)
        plsc.subcore_barrier()
        pltpu.sync_copy(acc.at[pl.ds(sid * 1024, 1024)],
                        out_h.at[cid, pl.ds(sid * 1024, 1024)])

    fn = pl.kernel(
        body,
        out_type=jax.ShapeDtypeStruct((NC, AW), f32),
        mesh=_sc_mesh(),
        compiler_params=pltpu.CompilerParams(use_tc_tiling_on_sc=False),
        scratch_types=[
            pltpu.VMEM_SHARED((AW,), f32),
            pltpu.VMEM((EPS,), i32), pltpu.VMEM((EPS,), i32),
            pltpu.VMEM((EPS,), i32), pltpu.VMEM((EPS,), i32),
            pltpu.VMEM((EPS,), f32),
            pltpu.VMEM((CH,), i32),
            pltpu.VMEM((CH,), f32),
            pltpu.VMEM((1024,), f32),
            pltpu.VMEM((L,), f32),
        ],
    )
    return fn(edst, esrc, sdst, ssrc, w)


def _sc_build_adj(edst, esrc, tok):
    EPS = EE // NS
    CH = 128
    NCHK = EPS // CH
    SLABW = 256 * NN
    NSLAB = (NN * NN) // SLABW
    ZW = 16384

    def body(edst_h, esrc_h, tok_h, out_h,
             acc, dstv, srcv, idxb, ones_b, zv, drb):
        cid = lax.axis_index("c")
        sid = lax.axis_index("s")
        base = sid * EPS
        pltpu.sync_copy(tok_h, drb)
        pltpu.sync_copy(edst_h.at[pl.ds(base, EPS)], dstv)
        pltpu.sync_copy(esrc_h.at[pl.ds(base, EPS)], srcv)
        _zero_vmem_1d(zv, ZW)

        def fill_ones(e, _):
            ones_b[pl.ds(e * L, L)] = jnp.full((L,), 1.0, f32)
            return 0
        lax.fori_loop(0, CH // L, fill_ones, 0)

        def do_slab(t, _):
            slab = 2 * t + cid
            word0 = slab * SLABW
            for z in range(SLABW // NS // ZW):
                pltpu.sync_copy(zv, acc.at[pl.ds(sid * (SLABW // NS) + z * ZW, ZW)])
            @pl.when(sid == 0)
            def _():
                pltpu.sync_copy(zv.at[pl.ds(0, L)], acc.at[pl.ds(SLABW, L)])
            plsc.subcore_barrier()

            def do_chunk(c, _):
                for g in range(CH // L):
                    d = dstv[pl.ds(c * CH + g * L, L)]
                    s = srcv[pl.ds(c * CH + g * L, L)]
                    loc = d * NN + s - word0
                    ok = (loc >= 0) & (loc < SLABW)
                    idxb[pl.ds(g * L, L)] = jnp.where(ok, loc, SLABW)
                pltpu.sync_copy(ones_b, acc.at[idxb], add=True)
                return 0
            lax.fori_loop(0, NCHK, do_chunk, 0)
            pltpu.sync_copy(zv.at[pl.ds(0, CH)], acc.at[idxb], add=True)
            pltpu.sync_copy(acc.at[pl.ds(SLABW, L)], drb)
            plsc.subcore_barrier()
            pltpu.sync_copy(acc.at[pl.ds(sid * (SLABW // NS), SLABW // NS)],
                            out_h.at[pl.ds(word0 + sid * (SLABW // NS), SLABW // NS)])
            plsc.subcore_barrier()
            return 0
        lax.fori_loop(0, NSLAB // NC, do_slab, 0)

    fn = pl.kernel(
        body,
        out_type=jax.ShapeDtypeStruct((NN * NN,), f32),
        mesh=_sc_mesh(),
        compiler_params=pltpu.CompilerParams(use_tc_tiling_on_sc=False),
        scratch_types=[
            pltpu.VMEM_SHARED((SLABW + L,), f32),
            pltpu.VMEM((EPS,), i32), pltpu.VMEM((EPS,), i32),
            pltpu.VMEM((CH,), i32),
            pltpu.VMEM((CH,), f32),
            pltpu.VMEM((ZW,), f32),
            pltpu.VMEM((L,), f32),
        ],
    )
    return fn(edst, esrc, tok)


def _sc_prop(table2, gat, sct, w, tok):
    _, n, d2 = table2.shape
    EPS = EE // NS
    CH = 128
    NCHK = EPS // CH
    RPS = NN // NS
    weighted = w is not None

    def body(*refs):
        if weighted:
            (tab_h, gat_h, sct_h, w_h, tok_h, out_h,
             acc, gidx, sidx, rows0, rows1, zrows, drb, tkv, wv, sem) = refs
        else:
            (tab_h, gat_h, sct_h, tok_h, out_h,
             acc, gidx, sidx, rows0, rows1, zrows, drb, tkv, sem) = refs
        rows = (rows0, rows1)
        cid = lax.axis_index("c")
        sid = lax.axis_index("s")
        pltpu.sync_copy(tok_h, tkv)
        pltpu.sync_copy(gat_h.at[pl.ds(sid * NCHK, NCHK)], gidx)
        pltpu.sync_copy(sct_h.at[pl.ds(sid * NCHK, NCHK)], sidx)
        if weighted:
            pltpu.sync_copy(w_h.at[pl.ds(sid * EPS, EPS)], wv.at[pl.ds(0, EPS)])
        _zero_vmem_rows(zrows, CH, d2)
        for z in range(RPS // CH):
            pltpu.sync_copy(zrows, acc.at[pl.ds(sid * RPS + z * CH, CH)])
        plsc.subcore_barrier()
        pending = pltpu.async_copy(tab_h.at[cid].at[gidx.at[0]], rows[0], sem)
        for c in range(NCHK):
            pending.wait()
            if c + 1 < NCHK:
                pending = pltpu.async_copy(
                    tab_h.at[cid].at[gidx.at[c + 1]], rows[(c + 1) % 2], sem)
            rb = rows[c % 2]
            if weighted:
                def scale(e, _):
                    ws = wv[pl.ds(c * CH + e, L)][0]
                    for g in range(d2 // L):
                        rb[e, pl.ds(g * L, L)] = rb[e, pl.ds(g * L, L)] * ws
                    return 0
                lax.fori_loop(0, CH, scale, 0)
            pltpu.sync_copy(rb, acc.at[sidx.at[c]], add=True)
        pltpu.sync_copy(zrows, acc.at[sidx.at[NCHK - 1]], add=True)
        pltpu.sync_copy(acc.at[pl.ds(0, 1)], drb)
        plsc.subcore_barrier()
        pltpu.sync_copy(acc.at[pl.ds(sid * RPS, RPS)],
                        out_h.at[cid, pl.ds(sid * RPS, RPS)])

    scratch = [
        pltpu.VMEM_SHARED((NN, d2), f32),
        pltpu.VMEM((NCHK, CH), i32),
        pltpu.VMEM((NCHK, CH), i32),
        pltpu.VMEM((CH, d2), f32),
        pltpu.VMEM((CH, d2), f32),
        pltpu.VMEM((CH, d2), f32),
        pltpu.VMEM((1, d2), f32),
        pltpu.VMEM((L,), f32),
    ]
    if weighted:
        scratch.append(pltpu.VMEM((EPS + L,), f32))
    scratch.append(pltpu.SemaphoreType.DMA)
    fn = pl.kernel(
        body,
        out_type=jax.ShapeDtypeStruct((NC, NN, d2), f32),
        mesh=_sc_mesh(),
        compiler_params=pltpu.CompilerParams(use_tc_tiling_on_sc=False),
        scratch_types=scratch,
    )
    args = (table2, gat.reshape(-1, CH), sct.reshape(-1, CH))
    args += ((w,) if weighted else ()) + (tok,)
    return fn(*args)


def _halves(t):
    d2 = t.shape[1] // 2
    return jnp.stack([t[:, :d2], t[:, d2:]])


def _unhalves(p):
    return jnp.concatenate([p[0], p[1]], axis=1)


def _bases_2d(x, gt):
    g = [gt[i][None, :] for i in range(12)]
    B = [jnp.where((x >= g[i]) & (x < g[i + 1]), 1.0, 0.0).astype(f32)
         for i in range(11)]
    for j in range(1, 4):
        B = [(x - g[i]) / (g[i + j] - g[i]) * B[i]
             + (g[i + j + 1] - x) / (g[i + j + 1] - g[i + 1]) * B[i + 1]
             for i in range(len(B) - 1)]
    return B


def _kan_compute(x, gt, bw, sw_ref):
    silu = x * jax.nn.sigmoid(x)
    acc = lax.dot_general(silu, bw, (((1,), (1,)), ((), ())),
                          preferred_element_type=f32)
    for j, Bj in enumerate(_bases_2d(x, gt)):
        acc += lax.dot_general(Bj, sw_ref[j], (((1,), (1,)), ((), ())),
                               preferred_element_type=f32)
    return acc


def _kan(parts, pre, p, posts, blk=256):
    n, din = parts[0].shape
    dout = p['base_w'].shape[0]
    nparts, npost = len(parts), len(posts)
    has_pre = pre is not None
    gt = jnp.transpose(p['grid'])
    sw8 = jnp.transpose(p['spline_w'], (2, 0, 1))

    def kbody(*refs):
        k = 0
        x = refs[0][...]
        for t in range(1, nparts):
            x = x + refs[t][...]
        k = nparts
        if has_pre:
            x = x * refs[k][...]
            k += 1
        gt_v = refs[k][...]; k += 1
        bw = refs[k][...]; k += 1
        sw_ref = refs[k]; k += 1
        post_refs = refs[k:k + npost]; k += npost
        out_refs = refs[k:]
        h = _kan_compute(x, gt_v, bw, sw_ref)
        if npost == 0:
            out_refs[0][...] = h
        else:
            for t in range(npost):
                out_refs[t][...] = post_refs[t][...] * h

    grid = (n // blk,)
    rowspec = pl.BlockSpec((blk, din), lambda i: (i, 0))
    vecspec = pl.BlockSpec((blk, 1), lambda i: (i, 0))
    in_specs = [rowspec] * nparts
    args = list(parts)
    if has_pre:
        in_specs.append(vecspec); args.append(pre)
    in_specs += [pl.BlockSpec((12, din), lambda i: (0, 0)),
                 pl.BlockSpec((dout, din), lambda i: (0, 0)),
                 pl.BlockSpec((8, dout, din), lambda i: (0, 0, 0))]
    args += [gt, p['base_w'], sw8]
    in_specs += [vecspec] * npost
    args += list(posts)
    nout = max(npost, 1)
    out = pl.pallas_call(
        kbody, grid=grid, in_specs=in_specs,
        out_specs=[pl.BlockSpec((blk, dout), lambda i: (i, 0))] * nout,
        out_shape=[jax.ShapeDtypeStruct((n, dout), f32)] * nout,
    )(*args)
    return out


def _tc_matmul_A2(abf):
    bm = bn = bk = 512
    I, J, K = NN // bm, NN // bn, NN // bk

    def body(l_ref, r_ref, o_ref, acc):
        i, j, k = pl.program_id(0), pl.program_id(1), pl.program_id(2)
        @pl.when(k == 0)
        def _():
            acc[...] = jnp.zeros((bm, bn), f32)
        acc[...] += lax.dot_general(l_ref[...], r_ref[...],
                                    (((1,), (0,)), ((), ())),
                                    preferred_element_type=f32)
        @pl.when(k == K - 1)
        def _():
            ri = lax.broadcasted_iota(i32, (bm, bn), 0) + i * bm
            ci = lax.broadcasted_iota(i32, (bm, bn), 1) + j * bn
            o_ref[...] = jnp.where(ri == ci, 0.0, acc[...]).astype(jnp.bfloat16)

    return pl.pallas_call(
        body, grid=(I, J, K),
        in_specs=[pl.BlockSpec((bm, bk), lambda i, j, k: (i, k)),
                  pl.BlockSpec((bk, bn), lambda i, j, k: (k, j))],
        out_specs=pl.BlockSpec((bm, bn), lambda i, j, k: (i, j)),
        out_shape=jax.ShapeDtypeStruct((NN, NN), jnp.bfloat16),
        scratch_shapes=[pltpu.VMEM((bm, bn), f32)],
        compiler_params=pltpu.CompilerParams(
            dimension_semantics=("parallel", "parallel", "arbitrary")),
    )(abf, abf)


def _tc_degsum(m):
    bm = 512
    I = NN // bm

    def body(m_ref, rs_ref, cs_ref):
        blk = m_ref[...].astype(f32)
        rs_ref[...] = jnp.sum(blk, axis=1, keepdims=True)
        cs_ref[...] = jnp.sum(blk, axis=0, keepdims=True)[None]

    rs, cs = pl.pallas_call(
        body, grid=(I,),
        in_specs=[pl.BlockSpec((bm, NN), lambda i: (i, 0))],
        out_specs=[pl.BlockSpec((bm, 1), lambda i: (i, 0)),
                   pl.BlockSpec((1, 1, NN), lambda i: (i, 0, 0))],
        out_shape=[jax.ShapeDtypeStruct((NN, 1), f32),
                   jax.ShapeDtypeStruct((I, 1, NN), f32)],
    )(m)
    return rs, cs.reshape(I, NN)


def _tc_finalize_sp(degs_t):
    def body(d_ref, o_ref):
        d = d_ref[...]
        def rs_of(col):
            v = d[:, col:col + 1] + d[:, col + 4:col + 5]
            return lax.rsqrt(jnp.maximum(v, 1e-12))
        b = rs_of(0)
        a = rs_of(1)
        bs = rs_of(2)
        a_s = rs_of(3)
        o_ref[...] = jnp.concatenate([a, b, a_s, bs], axis=1)

    return pl.pallas_call(
        body,
        in_specs=[pl.BlockSpec((NN, 8), lambda: (0, 0))],
        out_specs=pl.BlockSpec((NN, 4), lambda: (0, 0)),
        out_shape=jax.ShapeDtypeStruct((NN, 4), f32),
    )(degs_t)


def _tc_finalize_dn(rowsum, colsum_t):
    nI = colsum_t.shape[1]

    def body(r_ref, c_ref, o_ref):
        rd = lax.rsqrt(jnp.maximum(r_ref[...], 1e-12))
        cs = jnp.sum(c_ref[...], axis=1, keepdims=True)
        rs = lax.rsqrt(jnp.maximum(cs, 1e-12))
        o_ref[...] = jnp.concatenate([rd, rs], axis=1)

    return pl.pallas_call(
        body,
        in_specs=[pl.BlockSpec((NN, 1), lambda: (0, 0)),
                  pl.BlockSpec((NN, nI), lambda: (0, 0))],
        out_specs=pl.BlockSpec((NN, 2), lambda: (0, 0)),
        out_shape=jax.ShapeDtypeStruct((NN, 2), f32),
    )(rowsum, colsum_t)


def _mm_prop(m, u, trans):
    n, d = u.shape
    bm, bk = 512, 512
    I, K = NN // bm, NN // bk

    def body(l_ref, r_ref, o_ref, acc):
        k = pl.program_id(1)
        @pl.when(k == 0)
        def _():
            acc[...] = jnp.zeros((bm, d), f32)
        dn = (((0,), (0,)), ((), ())) if trans else (((1,), (0,)), ((), ()))
        acc[...] += lax.dot_general(l_ref[...].astype(f32), r_ref[...], dn,
                                    preferred_element_type=f32)
        @pl.when(k == K - 1)
        def _():
            o_ref[...] = acc[...]

    if trans:
        lspec = pl.BlockSpec((bk, bm), lambda i, k: (k, i))
    else:
        lspec = pl.BlockSpec((bm, bk), lambda i, k: (i, k))
    return pl.pallas_call(
        body, grid=(I, K),
        in_specs=[lspec, pl.BlockSpec((bk, d), lambda i, k: (k, 0))],
        out_specs=pl.BlockSpec((bm, d), lambda i, k: (i, 0)),
        out_shape=jax.ShapeDtypeStruct((NN, d), f32),
        scratch_shapes=[pltpu.VMEM((bm, d), f32)],
        compiler_params=pltpu.CompilerParams(
            dimension_semantics=("parallel", "arbitrary")),
    )(m, u)


def _fu_assemble(fp, feat_cfgs, blk=512):
    d = feat_cfgs[0][0][0].shape[1]
    att = fp['att']
    npart = [len(c[0]) for c in feat_cfgs]

    def body(*refs):
        i = pl.program_id(0)
        k = 0
        feats = []
        for t in range(3):
            x = refs[k][...]
            for _ in range(1, npart[t]):
                k += 1
                x = x + refs[k][...]
            k += 1
            x = x * refs[k][...]
            k += 1
            feats.append(x)
        lng = refs[k][...]; k += 1
        lnb = refs[k][...]; k += 1
        W = refs[k][...]; k += 1
        bv = refs[k][...]; k += 1
        qv = refs[k][...]; k += 1
        F_ref, wf_ref, cs_ref = refs[k], refs[k + 1], refs[k + 2]

        logits = []
        for t in range(3):
            f = feats[t]
            m = jnp.mean(f, axis=1, keepdims=True)
            v = jnp.mean((f - m) ** 2, axis=1, keepdims=True)
            fn = (f - m) * lax.rsqrt(v + 1e-5) * lng + lnb
            tt = jnp.tanh(lax.dot_general(fn, W, (((1,), (1,)), ((), ())),
                                          preferred_element_type=f32) + bv)
            logits.append(lax.dot_general(tt, qv, (((1,), (1,)), ((), ())),
                                          preferred_element_type=f32))
        mx = jnp.maximum(jnp.maximum(logits[0], logits[1]), logits[2])
        es = [jnp.exp(lg - mx) for lg in logits]
        den = es[0] + es[1] + es[2]
        wf = (es[0] * feats[0] + es[1] * feats[1] + es[2] * feats[2]) / den
        F = jnp.concatenate(feats, axis=1)
        F_ref[...] = F
        wf_ref[...] = wf
        @pl.when(i == 0)
        def _():
            cs_ref[...] = jnp.zeros((1, 3 * d), f32)
        cs_ref[...] += jnp.sum(F, axis=0, keepdims=True)

    grid = (NN // blk,)
    rowspec = pl.BlockSpec((blk, d), lambda i: (i, 0))
    vecspec = pl.BlockSpec((blk, 1), lambda i: (i, 0))
    in_specs, args = [], []
    for parts, scale in feat_cfgs:
        in_specs += [rowspec] * len(parts) + [vecspec]
        args += list(parts) + [scale]
    in_specs += [pl.BlockSpec((1, d), lambda i: (0, 0))] * 2
    args += [att['ln_g'][None, :], att['ln_b'][None, :]]
    in_specs += [pl.BlockSpec((d, d), lambda i: (0, 0))]
    args += [att['W']]
    in_specs += [pl.BlockSpec((1, d), lambda i: (0, 0))] * 2
    args += [att['b'][None, :], att['q'][None, :]]
    return pl.pallas_call(
        body, grid=grid, in_specs=in_specs,
        out_specs=[pl.BlockSpec((blk, 3 * d), lambda i: (i, 0)),
                   pl.BlockSpec((blk, d), lambda i: (i, 0)),
                   pl.BlockSpec((1, 3 * d), lambda i: (0, 0))],
        out_shape=[jax.ShapeDtypeStruct((NN, 3 * d), f32),
                   jax.ShapeDtypeStruct((NN, d), f32),
                   jax.ShapeDtypeStruct((1, 3 * d), f32)],
    )(*args)


def _fu_sumsq(F, cs, blk=512):
    dcols = F.shape[1]

    def body(f_ref, c_ref, o_ref):
        i = pl.program_id(0)
        m = c_ref[...] * (1.0 / NN)
        dev = f_ref[...] - m
        @pl.when(i == 0)
        def _():
            o_ref[...] = jnp.zeros((1, dcols), f32)
        o_ref[...] += jnp.sum(dev * dev, axis=0, keepdims=True)

    return pl.pallas_call(
        body, grid=(NN // blk,),
        in_specs=[pl.BlockSpec((blk, dcols), lambda i: (i, 0)),
                  pl.BlockSpec((1, dcols), lambda i: (0, 0))],
        out_specs=pl.BlockSpec((1, dcols), lambda i: (0, 0)),
        out_shape=jax.ShapeDtypeStruct((1, dcols), f32),
    )(F, cs)


def _fu_kan(F, cs, ss, fp, blk=256):
    dcols = F.shape[1]
    p = fp['kan']
    dout = p['base_w'].shape[0]
    gt = jnp.transpose(p['grid'])
    sw8 = jnp.transpose(p['spline_w'], (2, 0, 1))

    def body(f_ref, c_ref, s_ref, g_ref, b_ref, gt_ref, bw_ref, sw_ref,
             o_ref, oc_ref):
        i = pl.program_id(0)
        m = c_ref[...] * (1.0 / NN)
        v = s_ref[...] * (1.0 / NN)
        xb = (f_ref[...] - m) * lax.rsqrt(v + 1e-5) * g_ref[...] + b_ref[...]
        h = _kan_compute(xb, gt_ref[...], bw_ref[...], sw_ref)
        o_ref[...] = h
        @pl.when(i == 0)
        def _():
            oc_ref[...] = jnp.zeros((1, dout), f32)
        oc_ref[...] += jnp.sum(h, axis=0, keepdims=True)

    cspec = pl.BlockSpec((1, dcols), lambda i: (0, 0))
    return pl.pallas_call(
        body, grid=(NN // blk,),
        in_specs=[pl.BlockSpec((blk, dcols), lambda i: (i, 0)), cspec, cspec,
                  cspec, cspec,
                  pl.BlockSpec((12, dcols), lambda i: (0, 0)),
                  pl.BlockSpec((dout, dcols), lambda i: (0, 0)),
                  pl.BlockSpec((8, dout, dcols), lambda i: (0, 0, 0))],
        out_specs=[pl.BlockSpec((blk, dout), lambda i: (i, 0)),
                   pl.BlockSpec((1, dout), lambda i: (0, 0))],
        out_shape=[jax.ShapeDtypeStruct((NN, dout), f32),
                   jax.ShapeDtypeStruct((1, dout), f32)],
    )(F, cs, ss, fp['bn1_g'][None, :], fp['bn1_b'][None, :], gt,
      p['base_w'], sw8)


def _fu_final(wf, kout, kcs, kss, fp, blk=512):
    d = kout.shape[1]

    def body(w_ref, k_ref, c_ref, s_ref, g_ref, b_ref, o_ref):
        m = c_ref[...] * (1.0 / NN)
        v = s_ref[...] * (1.0 / NN)
        h = (k_ref[...] - m) * lax.rsqrt(v + 1e-5) * g_ref[...] + b_ref[...]
        o_ref[...] = jnp.concatenate([w_ref[...], h], axis=1)

    cspec = pl.BlockSpec((1, d), lambda i: (0, 0))
    return pl.pallas_call(
        body, grid=(NN // blk,),
        in_specs=[pl.BlockSpec((blk, d), lambda i: (i, 0)),
                  pl.BlockSpec((blk, d), lambda i: (i, 0)),
                  cspec, cspec, cspec, cspec],
        out_specs=pl.BlockSpec((blk, 2 * d), lambda i: (i, 0)),
        out_shape=jax.ShapeDtypeStruct((NN, 2 * d), f32),
    )(wf, kout, kcs, kss, fp['bn2_g'][None, :], fp['bn2_b'][None, :])


def _fusion(fp, f1, f2, f3):
    F, wf, cs = _fu_assemble(fp, [f1, f2, f3])
    ss = _fu_sumsq(F, cs)
    kout, kcs = _fu_kan(F, cs, ss, fp)
    kss = _fu_sumsq(kout, kcs)
    return _fu_final(wf, kout, kcs, kss, fp)


def kernel(x, edge_index, sim_index, sim_weight, params):
    src, dst = edge_index[0], edge_index[1]
    ssrc, sdst = sim_index[0], sim_index[1]

    degs = _sc_degrees(dst, src, sdst, ssrc, sim_weight)
    degs_t = jnp.transpose(degs.reshape(8, NN))
    Aflat = _sc_build_adj(dst, src, degs[0, :L])
    A = Aflat.reshape(NN, NN)
    A2 = _tc_matmul_A2(A.astype(jnp.bfloat16))
    rowsum, colsum_part = _tc_degsum(A2)
    scales = _tc_finalize_sp(degs_t)
    a_ = scales[:, 0:1]
    b_ = scales[:, 1:2]
    as_ = scales[:, 2:3]
    bs_ = scales[:, 3:4]
    scales2 = _tc_finalize_dn(rowsum, jnp.transpose(colsum_part))
    rd = scales2[:, 0:1]
    rs = scales2[:, 1:2]

    pdd, pco, psm = params['ddi'], params['co'], params['sim']
    t_in, t_out = _kan([x], None, pdd['kan1'], [a_, b_])
    t_sim = _kan([x], None, psm['kan1'], [as_])[0]
    u_in, u_out = _kan([x], None, pco['kan1'], [rs, rd])

    P_in = _sc_prop(_halves(t_in), src, dst, None, Aflat[:L])
    P_out = _sc_prop(_halves(t_out), dst, src, None, P_in[0, 0, :L])
    S1 = _sc_prop(_halves(t_sim), ssrc, sdst, sim_weight, P_out[0, 0, :L])
    Y_in = _mm_prop(A2, u_in, False)
    Y_out = _mm_prop(A2, u_out, True)

    t2_in, = _kan([_unhalves(P_in)], b_, pdd['kan2'], [a_])
    t2_out, = _kan([_unhalves(P_out)], a_, pdd['kan2'], [b_])
    t2_sim, = _kan([_unhalves(S1)], bs_, psm['kan2'], [as_])
    u2_in, = _kan([Y_in], rd, pco['kan2'], [rs])
    u2_out, = _kan([Y_out], rs, pco['kan2'], [rd])

    X_in = _sc_prop(_halves(t2_in), src, dst, None, S1[0, 0, :L])
    X_out = _sc_prop(_halves(t2_out), dst, src, None, X_in[0, 0, :L])
    S2 = _sc_prop(_halves(t2_sim), ssrc, sdst, sim_weight, X_out[0, 0, :L])
    Y2_in = _mm_prop(A2, u2_in, False)
    Y2_out = _mm_prop(A2, u2_out, True)

    x_sim_f = _unhalves(S2)
    x_in = _fusion(params['in_fusion'],
                   ([_unhalves(X_in)], b_), ([Y2_in], rd), ([x_sim_f], bs_))
    x_out = _fusion(params['out_fusion'],
                    ([_unhalves(X_out)], a_), ([Y2_out], rs), ([x_sim_f], bs_))
    return (x_in, x_out)


# speedup vs baseline: 1.2912x; 1.0018x over previous
import functools

import jax
import jax.numpy as jnp
from jax import lax
from jax.experimental import pallas as pl
from jax.experimental.pallas import tpu as pltpu
from jax.experimental.pallas import tpu_sc as plsc

NN = 4096
EE = 65536
NC, NS, L = 2, 16, 16

f32 = jnp.float32
i32 = jnp.int32


def _sc_mesh():
    return plsc.VectorSubcoreMesh(core_axis_name="c", subcore_axis_name="s")


def _zero_vmem_rows(ref, nrows, width):
    def body(e, _):
        for g in range(width // L):
            ref[e, pl.ds(g * L, L)] = jnp.zeros((L,), f32)
        return 0
    lax.fori_loop(0, nrows, body, 0)


def _zero_vmem_1d(ref, n):
    def body(e, _):
        ref[pl.ds(e * L, L)] = jnp.zeros((L,), f32)
        return 0
    lax.fori_loop(0, n // L, body, 0)


def _sc_degrees(edst, esrc, sdst, ssrc, w):
    EPS = EE // (NC * NS)
    CH = 128
    NCHK = EPS // CH
    AW = 4 * NN

    def body(edst_h, esrc_h, sdst_h, ssrc_h, w_h, out_h,
             acc, dstv, srcv, sdv, ssv, wv, idxb, ones_b, zv, drb):
        cid = lax.axis_index("c")
        sid = lax.axis_index("s")
        base = (cid * NS + sid) * EPS
        pltpu.sync_copy(edst_h.at[pl.ds(base, EPS)], dstv)
        pltpu.sync_copy(esrc_h.at[pl.ds(base, EPS)], srcv)
        pltpu.sync_copy(sdst_h.at[pl.ds(base, EPS)], sdv)
        pltpu.sync_copy(ssrc_h.at[pl.ds(base, EPS)], ssv)
        pltpu.sync_copy(w_h.at[pl.ds(base, EPS)], wv)
        _zero_vmem_1d(zv, 1024)

        def fill_ones(e, _):
            ones_b[pl.ds(e * L, L)] = jnp.full((L,), 1.0, f32)
            return 0
        lax.fori_loop(0, CH // L, fill_ones, 0)

        pltpu.sync_copy(zv, acc.at[pl.ds(sid * 1024, 1024)])
        plsc.subcore_barrier()

        for c in range(NCHK):
            for (vec, off, vals) in (
                    (dstv, 0, ones_b), (srcv, NN, ones_b),
                    (sdv, 2 * NN, wv.at[pl.ds(c * CH, CH)]),
                    (ssv, 3 * NN, wv.at[pl.ds(c * CH, CH)])):
                for g in range(CH // L):
                    v = vec[pl.ds(c * CH + g * L, L)]
                    idxb[pl.ds(g * L, L)] = v + off
                pltpu.sync_copy(vals, acc.at[idxb], add=True)
        pltpu.sync_copy(zv.at[pl.ds(0, CH)], acc.at[idxb], add=True)
        pltpu.sync_copy(acc.at[pl.ds(0, L)], drb)
        plsc.subcore_barrier()
        pltpu.sync_copy(acc.at[pl.ds(sid * 1024, 1024)],
                        out_h.at[cid, pl.ds(sid * 1024, 1024)])

    fn = pl.kernel(
        body,
        out_type=jax.ShapeDtypeStruct((NC, AW), f32),
        mesh=_sc_mesh(),
        compiler_params=pltpu.CompilerParams(use_tc_tiling_on_sc=False),
        scratch_types=[
            pltpu.VMEM_SHARED((AW,), f32),
            pltpu.VMEM((EPS,), i32), pltpu.VMEM((EPS,), i32),
            pltpu.VMEM((EPS,), i32), pltpu.VMEM((EPS,), i32),
            pltpu.VMEM((EPS,), f32),
            pltpu.VMEM((CH,), i32),
            pltpu.VMEM((CH,), f32),
            pltpu.VMEM((1024,), f32),
            pltpu.VMEM((L,), f32),
        ],
    )
    return fn(edst, esrc, sdst, ssrc, w)


def _sc_build_adj(edst, esrc, tok):
    EPS = EE // NS
    CH = 128
    NCHK = EPS // CH
    SLABW = 256 * NN
    NSLAB = (NN * NN) // SLABW
    ZW = 16384

    def body(edst_h, esrc_h, tok_h, out_h,
             acc, dstv, srcv, idxb, ones_b, zv, drb):
        cid = lax.axis_index("c")
        sid = lax.axis_index("s")
        base = sid * EPS
        pltpu.sync_copy(tok_h, drb)
        pltpu.sync_copy(edst_h.at[pl.ds(base, EPS)], dstv)
        pltpu.sync_copy(esrc_h.at[pl.ds(base, EPS)], srcv)
        _zero_vmem_1d(zv, ZW)

        def fill_ones(e, _):
            ones_b[pl.ds(e * L, L)] = jnp.full((L,), 1.0, f32)
            return 0
        lax.fori_loop(0, CH // L, fill_ones, 0)

        def do_slab(t, _):
            slab = 2 * t + cid
            word0 = slab * SLABW
            for z in range(SLABW // NS // ZW):
                pltpu.sync_copy(zv, acc.at[pl.ds(sid * (SLABW // NS) + z * ZW, ZW)])
            @pl.when(sid == 0)
            def _():
                pltpu.sync_copy(zv.at[pl.ds(0, L)], acc.at[pl.ds(SLABW, L)])
            plsc.subcore_barrier()

            def do_chunk(c, _):
                for g in range(CH // L):
                    d = dstv[pl.ds(c * CH + g * L, L)]
                    s = srcv[pl.ds(c * CH + g * L, L)]
                    loc = d * NN + s - word0
                    ok = (loc >= 0) & (loc < SLABW)
                    idxb[pl.ds(g * L, L)] = jnp.where(ok, loc, SLABW)
                pltpu.sync_copy(ones_b, acc.at[idxb], add=True)
                return 0
            lax.fori_loop(0, NCHK, do_chunk, 0)
            pltpu.sync_copy(zv.at[pl.ds(0, CH)], acc.at[idxb], add=True)
            pltpu.sync_copy(acc.at[pl.ds(SLABW, L)], drb)
            plsc.subcore_barrier()
            pltpu.sync_copy(acc.at[pl.ds(sid * (SLABW // NS), SLABW // NS)],
                            out_h.at[pl.ds(word0 + sid * (SLABW // NS), SLABW // NS)])
            plsc.subcore_barrier()
            return 0
        lax.fori_loop(0, NSLAB // NC, do_slab, 0)

    fn = pl.kernel(
        body,
        out_type=jax.ShapeDtypeStruct((NN * NN,), f32),
        mesh=_sc_mesh(),
        compiler_params=pltpu.CompilerParams(use_tc_tiling_on_sc=False),
        scratch_types=[
            pltpu.VMEM_SHARED((SLABW + L,), f32),
            pltpu.VMEM((EPS,), i32), pltpu.VMEM((EPS,), i32),
            pltpu.VMEM((CH,), i32),
            pltpu.VMEM((CH,), f32),
            pltpu.VMEM((ZW,), f32),
            pltpu.VMEM((L,), f32),
        ],
    )
    return fn(edst, esrc, tok)


def _sc_prop(table2, gat, sct, w, tok):
    _, n, d2 = table2.shape
    EPS = EE // NS
    CH = 128
    NCHK = EPS // CH
    RPS = NN // NS
    weighted = w is not None

    def body(*refs):
        if weighted:
            (tab_h, gat_h, sct_h, w_h, tok_h, out_h,
             acc, gidx, sidx, rows0, rows1, zrows, drb, tkv, wv, sem) = refs
        else:
            (tab_h, gat_h, sct_h, tok_h, out_h,
             acc, gidx, sidx, rows0, rows1, zrows, drb, tkv, sem) = refs
        rows = (rows0, rows1)
        cid = lax.axis_index("c")
        sid = lax.axis_index("s")
        pltpu.sync_copy(tok_h, tkv)
        pltpu.sync_copy(gat_h.at[pl.ds(sid * NCHK, NCHK)], gidx)
        pltpu.sync_copy(sct_h.at[pl.ds(sid * NCHK, NCHK)], sidx)
        if weighted:
            pltpu.sync_copy(w_h.at[pl.ds(sid * EPS, EPS)], wv.at[pl.ds(0, EPS)])
        _zero_vmem_rows(zrows, CH, d2)
        for z in range(RPS // CH):
            pltpu.sync_copy(zrows, acc.at[pl.ds(sid * RPS + z * CH, CH)])
        plsc.subcore_barrier()
        pending = pltpu.async_copy(tab_h.at[cid].at[gidx.at[0]], rows[0], sem)
        for c in range(NCHK):
            pending.wait()
            if c + 1 < NCHK:
                pending = pltpu.async_copy(
                    tab_h.at[cid].at[gidx.at[c + 1]], rows[(c + 1) % 2], sem)
            rb = rows[c % 2]
            if weighted:
                def scale(e, _):
                    ws = wv[pl.ds(c * CH + e, L)][0]
                    for g in range(d2 // L):
                        rb[e, pl.ds(g * L, L)] = rb[e, pl.ds(g * L, L)] * ws
                    return 0
                lax.fori_loop(0, CH, scale, 0)
            pltpu.sync_copy(rb, acc.at[sidx.at[c]], add=True)
        pltpu.sync_copy(zrows, acc.at[sidx.at[NCHK - 1]], add=True)
        pltpu.sync_copy(acc.at[pl.ds(0, 1)], drb)
        plsc.subcore_barrier()
        pltpu.sync_copy(acc.at[pl.ds(sid * RPS, RPS)],
                        out_h.at[cid, pl.ds(sid * RPS, RPS)])

    scratch = [
        pltpu.VMEM_SHARED((NN, d2), f32),
        pltpu.VMEM((NCHK, CH), i32),
        pltpu.VMEM((NCHK, CH), i32),
        pltpu.VMEM((CH, d2), f32),
        pltpu.VMEM((CH, d2), f32),
        pltpu.VMEM((CH, d2), f32),
        pltpu.VMEM((1, d2), f32),
        pltpu.VMEM((L,), f32),
    ]
    if weighted:
        scratch.append(pltpu.VMEM((EPS + L,), f32))
    scratch.append(pltpu.SemaphoreType.DMA)
    fn = pl.kernel(
        body,
        out_type=jax.ShapeDtypeStruct((NC, NN, d2), f32),
        mesh=_sc_mesh(),
        compiler_params=pltpu.CompilerParams(use_tc_tiling_on_sc=False),
        scratch_types=scratch,
    )
    args = (table2, gat.reshape(-1, CH), sct.reshape(-1, CH))
    args += ((w,) if weighted else ()) + (tok,)
    return fn(*args)


def _halves(t):
    d2 = t.shape[1] // 2
    return jnp.stack([t[:, :d2], t[:, d2:]])


def _unhalves(p):
    return jnp.concatenate([p[0], p[1]], axis=1)


def _bases_2d(x, gt):
    g = [gt[i][None, :] for i in range(12)]
    B = [jnp.where((x >= g[i]) & (x < g[i + 1]), 1.0, 0.0).astype(f32)
         for i in range(11)]
    for j in range(1, 4):
        B = [(x - g[i]) / (g[i + j] - g[i]) * B[i]
             + (g[i + j + 1] - x) / (g[i + j + 1] - g[i + 1]) * B[i + 1]
             for i in range(len(B) - 1)]
    return B


def _kan_compute(x, gt, bw, sw_ref):
    silu = x * jax.nn.sigmoid(x)
    acc = lax.dot_general(silu, bw, (((1,), (1,)), ((), ())),
                          preferred_element_type=f32)
    for j, Bj in enumerate(_bases_2d(x, gt)):
        acc += lax.dot_general(Bj, sw_ref[j], (((1,), (1,)), ((), ())),
                               preferred_element_type=f32)
    return acc


def _kan(parts, pre, p, posts, blk=512):
    n, din = parts[0].shape
    dout = p['base_w'].shape[0]
    nparts, npost = len(parts), len(posts)
    has_pre = pre is not None
    gt = jnp.transpose(p['grid'])
    sw8 = jnp.transpose(p['spline_w'], (2, 0, 1))

    def kbody(*refs):
        k = 0
        x = refs[0][...]
        for t in range(1, nparts):
            x = x + refs[t][...]
        k = nparts
        if has_pre:
            x = x * refs[k][...]
            k += 1
        gt_v = refs[k][...]; k += 1
        bw = refs[k][...]; k += 1
        sw_ref = refs[k]; k += 1
        post_refs = refs[k:k + npost]; k += npost
        out_refs = refs[k:]
        h = _kan_compute(x, gt_v, bw, sw_ref)
        if npost == 0:
            out_refs[0][...] = h
        else:
            for t in range(npost):
                out_refs[t][...] = post_refs[t][...] * h

    grid = (n // blk,)
    rowspec = pl.BlockSpec((blk, din), lambda i: (i, 0))
    vecspec = pl.BlockSpec((blk, 1), lambda i: (i, 0))
    in_specs = [rowspec] * nparts
    args = list(parts)
    if has_pre:
        in_specs.append(vecspec); args.append(pre)
    in_specs += [pl.BlockSpec((12, din), lambda i: (0, 0)),
                 pl.BlockSpec((dout, din), lambda i: (0, 0)),
                 pl.BlockSpec((8, dout, din), lambda i: (0, 0, 0))]
    args += [gt, p['base_w'], sw8]
    in_specs += [vecspec] * npost
    args += list(posts)
    nout = max(npost, 1)
    out = pl.pallas_call(
        kbody, grid=grid, in_specs=in_specs,
        out_specs=[pl.BlockSpec((blk, dout), lambda i: (i, 0))] * nout,
        out_shape=[jax.ShapeDtypeStruct((n, dout), f32)] * nout,
    )(*args)
    return out


def _tc_matmul_A2(abf):
    bm = bn = bk = 512
    I, J, K = NN // bm, NN // bn, NN // bk

    def body(l_ref, r_ref, o_ref, acc):
        i, j, k = pl.program_id(0), pl.program_id(1), pl.program_id(2)
        @pl.when(k == 0)
        def _():
            acc[...] = jnp.zeros((bm, bn), f32)
        acc[...] += lax.dot_general(l_ref[...], r_ref[...],
                                    (((1,), (0,)), ((), ())),
                                    preferred_element_type=f32)
        @pl.when(k == K - 1)
        def _():
            ri = lax.broadcasted_iota(i32, (bm, bn), 0) + i * bm
            ci = lax.broadcasted_iota(i32, (bm, bn), 1) + j * bn
            o_ref[...] = jnp.where(ri == ci, 0.0, acc[...]).astype(jnp.bfloat16)

    return pl.pallas_call(
        body, grid=(I, J, K),
        in_specs=[pl.BlockSpec((bm, bk), lambda i, j, k: (i, k)),
                  pl.BlockSpec((bk, bn), lambda i, j, k: (k, j))],
        out_specs=pl.BlockSpec((bm, bn), lambda i, j, k: (i, j)),
        out_shape=jax.ShapeDtypeStruct((NN, NN), jnp.bfloat16),
        scratch_shapes=[pltpu.VMEM((bm, bn), f32)],
        compiler_params=pltpu.CompilerParams(
            dimension_semantics=("parallel", "parallel", "arbitrary")),
    )(abf, abf)


def _tc_degsum(m):
    bm = 512
    I = NN // bm

    def body(m_ref, rs_ref, cs_ref):
        blk = m_ref[...].astype(f32)
        rs_ref[...] = jnp.sum(blk, axis=1, keepdims=True)
        cs_ref[...] = jnp.sum(blk, axis=0, keepdims=True)[None]

    rs, cs = pl.pallas_call(
        body, grid=(I,),
        in_specs=[pl.BlockSpec((bm, NN), lambda i: (i, 0))],
        out_specs=[pl.BlockSpec((bm, 1), lambda i: (i, 0)),
                   pl.BlockSpec((1, 1, NN), lambda i: (i, 0, 0))],
        out_shape=[jax.ShapeDtypeStruct((NN, 1), f32),
                   jax.ShapeDtypeStruct((I, 1, NN), f32)],
    )(m)
    return rs, cs.reshape(I, NN)


def _tc_finalize_sp(degs_t):
    def body(d_ref, o_ref):
        d = d_ref[...]
        def rs_of(col):
            v = d[:, col:col + 1] + d[:, col + 4:col + 5]
            return lax.rsqrt(jnp.maximum(v, 1e-12))
        b = rs_of(0)
        a = rs_of(1)
        bs = rs_of(2)
        a_s = rs_of(3)
        o_ref[...] = jnp.concatenate([a, b, a_s, bs], axis=1)

    return pl.pallas_call(
        body,
        in_specs=[pl.BlockSpec((NN, 8), lambda: (0, 0))],
        out_specs=pl.BlockSpec((NN, 4), lambda: (0, 0)),
        out_shape=jax.ShapeDtypeStruct((NN, 4), f32),
    )(degs_t)


def _tc_finalize_dn(rowsum, colsum_t):
    nI = colsum_t.shape[1]

    def body(r_ref, c_ref, o_ref):
        rd = lax.rsqrt(jnp.maximum(r_ref[...], 1e-12))
        cs = jnp.sum(c_ref[...], axis=1, keepdims=True)
        rs = lax.rsqrt(jnp.maximum(cs, 1e-12))
        o_ref[...] = jnp.concatenate([rd, rs], axis=1)

    return pl.pallas_call(
        body,
        in_specs=[pl.BlockSpec((NN, 1), lambda: (0, 0)),
                  pl.BlockSpec((NN, nI), lambda: (0, 0))],
        out_specs=pl.BlockSpec((NN, 2), lambda: (0, 0)),
        out_shape=jax.ShapeDtypeStruct((NN, 2), f32),
    )(rowsum, colsum_t)


def _mm_prop(m, u, trans):
    n, d = u.shape
    bm, bk = 512, 512
    I, K = NN // bm, NN // bk

    def body(l_ref, r_ref, o_ref, acc):
        k = pl.program_id(1)
        @pl.when(k == 0)
        def _():
            acc[...] = jnp.zeros((bm, d), f32)
        dn = (((0,), (0,)), ((), ())) if trans else (((1,), (0,)), ((), ()))
        acc[...] += lax.dot_general(l_ref[...].astype(f32), r_ref[...], dn,
                                    preferred_element_type=f32)
        @pl.when(k == K - 1)
        def _():
            o_ref[...] = acc[...]

    if trans:
        lspec = pl.BlockSpec((bk, bm), lambda i, k: (k, i))
    else:
        lspec = pl.BlockSpec((bm, bk), lambda i, k: (i, k))
    return pl.pallas_call(
        body, grid=(I, K),
        in_specs=[lspec, pl.BlockSpec((bk, d), lambda i, k: (k, 0))],
        out_specs=pl.BlockSpec((bm, d), lambda i, k: (i, 0)),
        out_shape=jax.ShapeDtypeStruct((NN, d), f32),
        scratch_shapes=[pltpu.VMEM((bm, d), f32)],
        compiler_params=pltpu.CompilerParams(
            dimension_semantics=("parallel", "arbitrary")),
    )(m, u)


def _fu_assemble(fp, feat_cfgs, blk=512):
    d = feat_cfgs[0][0][0].shape[1]
    att = fp['att']
    npart = [len(c[0]) for c in feat_cfgs]

    def body(*refs):
        i = pl.program_id(0)
        k = 0
        feats = []
        for t in range(3):
            x = refs[k][...]
            for _ in range(1, npart[t]):
                k += 1
                x = x + refs[k][...]
            k += 1
            x = x * refs[k][...]
            k += 1
            feats.append(x)
        lng = refs[k][...]; k += 1
        lnb = refs[k][...]; k += 1
        W = refs[k][...]; k += 1
        bv = refs[k][...]; k += 1
        qv = refs[k][...]; k += 1
        F_ref, wf_ref, cs_ref = refs[k], refs[k + 1], refs[k + 2]

        logits = []
        for t in range(3):
            f = feats[t]
            m = jnp.mean(f, axis=1, keepdims=True)
            v = jnp.mean((f - m) ** 2, axis=1, keepdims=True)
            fn = (f - m) * lax.rsqrt(v + 1e-5) * lng + lnb
            tt = jnp.tanh(lax.dot_general(fn, W, (((1,), (1,)), ((), ())),
                                          preferred_element_type=f32) + bv)
            logits.append(lax.dot_general(tt, qv, (((1,), (1,)), ((), ())),
                                          preferred_element_type=f32))
        mx = jnp.maximum(jnp.maximum(logits[0], logits[1]), logits[2])
        es = [jnp.exp(lg - mx) for lg in logits]
        den = es[0] + es[1] + es[2]
        wf = (es[0] * feats[0] + es[1] * feats[1] + es[2] * feats[2]) / den
        F = jnp.concatenate(feats, axis=1)
        F_ref[...] = F
        wf_ref[...] = wf
        @pl.when(i == 0)
        def _():
            cs_ref[...] = jnp.zeros((1, 3 * d), f32)
        cs_ref[...] += jnp.sum(F, axis=0, keepdims=True)

    grid = (NN // blk,)
    rowspec = pl.BlockSpec((blk, d), lambda i: (i, 0))
    vecspec = pl.BlockSpec((blk, 1), lambda i: (i, 0))
    in_specs, args = [], []
    for parts, scale in feat_cfgs:
        in_specs += [rowspec] * len(parts) + [vecspec]
        args += list(parts) + [scale]
    in_specs += [pl.BlockSpec((1, d), lambda i: (0, 0))] * 2
    args += [att['ln_g'][None, :], att['ln_b'][None, :]]
    in_specs += [pl.BlockSpec((d, d), lambda i: (0, 0))]
    args += [att['W']]
    in_specs += [pl.BlockSpec((1, d), lambda i: (0, 0))] * 2
    args += [att['b'][None, :], att['q'][None, :]]
    return pl.pallas_call(
        body, grid=grid, in_specs=in_specs,
        out_specs=[pl.BlockSpec((blk, 3 * d), lambda i: (i, 0)),
                   pl.BlockSpec((blk, d), lambda i: (i, 0)),
                   pl.BlockSpec((1, 3 * d), lambda i: (0, 0))],
        out_shape=[jax.ShapeDtypeStruct((NN, 3 * d), f32),
                   jax.ShapeDtypeStruct((NN, d), f32),
                   jax.ShapeDtypeStruct((1, 3 * d), f32)],
    )(*args)


def _fu_sumsq(F, cs, blk=512):
    dcols = F.shape[1]

    def body(f_ref, c_ref, o_ref):
        i = pl.program_id(0)
        m = c_ref[...] * (1.0 / NN)
        dev = f_ref[...] - m
        @pl.when(i == 0)
        def _():
            o_ref[...] = jnp.zeros((1, dcols), f32)
        o_ref[...] += jnp.sum(dev * dev, axis=0, keepdims=True)

    return pl.pallas_call(
        body, grid=(NN // blk,),
        in_specs=[pl.BlockSpec((blk, dcols), lambda i: (i, 0)),
                  pl.BlockSpec((1, dcols), lambda i: (0, 0))],
        out_specs=pl.BlockSpec((1, dcols), lambda i: (0, 0)),
        out_shape=jax.ShapeDtypeStruct((1, dcols), f32),
    )(F, cs)


def _fu_kan(F, cs, ss, fp, blk=256):
    dcols = F.shape[1]
    p = fp['kan']
    dout = p['base_w'].shape[0]
    gt = jnp.transpose(p['grid'])
    sw8 = jnp.transpose(p['spline_w'], (2, 0, 1))

    def body(f_ref, c_ref, s_ref, g_ref, b_ref, gt_ref, bw_ref, sw_ref,
             o_ref, oc_ref):
        i = pl.program_id(0)
        m = c_ref[...] * (1.0 / NN)
        v = s_ref[...] * (1.0 / NN)
        xb = (f_ref[...] - m) * lax.rsqrt(v + 1e-5) * g_ref[...] + b_ref[...]
        h = _kan_compute(xb, gt_ref[...], bw_ref[...], sw_ref)
        o_ref[...] = h
        @pl.when(i == 0)
        def _():
            oc_ref[...] = jnp.zeros((1, dout), f32)
        oc_ref[...] += jnp.sum(h, axis=0, keepdims=True)

    cspec = pl.BlockSpec((1, dcols), lambda i: (0, 0))
    return pl.pallas_call(
        body, grid=(NN // blk,),
        in_specs=[pl.BlockSpec((blk, dcols), lambda i: (i, 0)), cspec, cspec,
                  cspec, cspec,
                  pl.BlockSpec((12, dcols), lambda i: (0, 0)),
                  pl.BlockSpec((dout, dcols), lambda i: (0, 0)),
                  pl.BlockSpec((8, dout, dcols), lambda i: (0, 0, 0))],
        out_specs=[pl.BlockSpec((blk, dout), lambda i: (i, 0)),
                   pl.BlockSpec((1, dout), lambda i: (0, 0))],
        out_shape=[jax.ShapeDtypeStruct((NN, dout), f32),
                   jax.ShapeDtypeStruct((1, dout), f32)],
    )(F, cs, ss, fp['bn1_g'][None, :], fp['bn1_b'][None, :], gt,
      p['base_w'], sw8)


def _fu_final(wf, kout, kcs, kss, fp, blk=512):
    d = kout.shape[1]

    def body(w_ref, k_ref, c_ref, s_ref, g_ref, b_ref, o_ref):
        m = c_ref[...] * (1.0 / NN)
        v = s_ref[...] * (1.0 / NN)
        h = (k_ref[...] - m) * lax.rsqrt(v + 1e-5) * g_ref[...] + b_ref[...]
        o_ref[...] = jnp.concatenate([w_ref[...], h], axis=1)

    cspec = pl.BlockSpec((1, d), lambda i: (0, 0))
    return pl.pallas_call(
        body, grid=(NN // blk,),
        in_specs=[pl.BlockSpec((blk, d), lambda i: (i, 0)),
                  pl.BlockSpec((blk, d), lambda i: (i, 0)),
                  cspec, cspec, cspec, cspec],
        out_specs=pl.BlockSpec((blk, 2 * d), lambda i: (i, 0)),
        out_shape=jax.ShapeDtypeStruct((NN, 2 * d), f32),
    )(wf, kout, kcs, kss, fp['bn2_g'][None, :], fp['bn2_b'][None, :])


def _fusion(fp, f1, f2, f3):
    F, wf, cs = _fu_assemble(fp, [f1, f2, f3])
    ss = _fu_sumsq(F, cs)
    kout, kcs = _fu_kan(F, cs, ss, fp)
    kss = _fu_sumsq(kout, kcs)
    return _fu_final(wf, kout, kcs, kss, fp)


def kernel(x, edge_index, sim_index, sim_weight, params):
    src, dst = edge_index[0], edge_index[1]
    ssrc, sdst = sim_index[0], sim_index[1]

    degs = _sc_degrees(dst, src, sdst, ssrc, sim_weight)
    degs_t = jnp.transpose(degs.reshape(8, NN))
    Aflat = _sc_build_adj(dst, src, degs[0, :L])
    A = Aflat.reshape(NN, NN)
    A2 = _tc_matmul_A2(A.astype(jnp.bfloat16))
    rowsum, colsum_part = _tc_degsum(A2)
    scales = _tc_finalize_sp(degs_t)
    a_ = scales[:, 0:1]
    b_ = scales[:, 1:2]
    as_ = scales[:, 2:3]
    bs_ = scales[:, 3:4]
    scales2 = _tc_finalize_dn(rowsum, jnp.transpose(colsum_part))
    rd = scales2[:, 0:1]
    rs = scales2[:, 1:2]

    pdd, pco, psm = params['ddi'], params['co'], params['sim']
    t_in, t_out = _kan([x], None, pdd['kan1'], [a_, b_])
    t_sim = _kan([x], None, psm['kan1'], [as_])[0]
    u_in, u_out = _kan([x], None, pco['kan1'], [rs, rd])

    P_in = _sc_prop(_halves(t_in), src, dst, None, Aflat[:L])
    P_out = _sc_prop(_halves(t_out), dst, src, None, P_in[0, 0, :L])
    S1 = _sc_prop(_halves(t_sim), ssrc, sdst, sim_weight, P_out[0, 0, :L])
    Y_in = _mm_prop(A2, u_in, False)
    Y_out = _mm_prop(A2, u_out, True)

    t2_in, = _kan([_unhalves(P_in)], b_, pdd['kan2'], [a_])
    t2_out, = _kan([_unhalves(P_out)], a_, pdd['kan2'], [b_])
    t2_sim, = _kan([_unhalves(S1)], bs_, psm['kan2'], [as_])
    u2_in, = _kan([Y_in], rd, pco['kan2'], [rs])
    u2_out, = _kan([Y_out], rs, pco['kan2'], [rd])

    X_in = _sc_prop(_halves(t2_in), src, dst, None, S1[0, 0, :L])
    X_out = _sc_prop(_halves(t2_out), dst, src, None, X_in[0, 0, :L])
    S2 = _sc_prop(_halves(t2_sim), ssrc, sdst, sim_weight, X_out[0, 0, :L])
    Y2_in = _mm_prop(A2, u2_in, False)
    Y2_out = _mm_prop(A2, u2_out, True)

    x_sim_f = _unhalves(S2)
    x_in = _fusion(params['in_fusion'],
                   ([_unhalves(X_in)], b_), ([Y2_in], rd), ([x_sim_f], bs_))
    x_out = _fusion(params['out_fusion'],
                    ([_unhalves(X_out)], a_), ([Y2_out], rs), ([x_sim_f], bs_))
    return (x_in, x_out)


# speedup vs baseline: 1.2923x; 1.0009x over previous
import jax
import jax.numpy as jnp
from jax import lax
from jax.experimental import pallas as pl
from jax.experimental.pallas import tpu as pltpu
from jax.experimental.pallas import tpu_sc as plsc

NN = 4096
EE = 65536
NC, NS, L = 2, 16, 16

f32 = jnp.float32
i32 = jnp.int32


def _sc_mesh():
    return plsc.VectorSubcoreMesh(core_axis_name="c", subcore_axis_name="s")


def _zero_vmem_rows(ref, nrows, width):
    def body(e, _):
        for g in range(width // L):
            ref[e, pl.ds(g * L, L)] = jnp.zeros((L,), f32)
        return 0
    lax.fori_loop(0, nrows, body, 0)


def _zero_vmem_1d(ref, n):
    def body(e, _):
        ref[pl.ds(e * L, L)] = jnp.zeros((L,), f32)
        return 0
    lax.fori_loop(0, n // L, body, 0)


def _sc_degrees(edst, esrc, sdst, ssrc, w):
    EPS = EE // (NC * NS)
    CH = 128
    NCHK = EPS // CH
    AW = 4 * NN

    def body(edst_h, esrc_h, sdst_h, ssrc_h, w_h, out_h,
             acc, dstv, srcv, sdv, ssv, wv, idxb, ones_b, zv, drb):
        cid = lax.axis_index("c")
        sid = lax.axis_index("s")
        base = (cid * NS + sid) * EPS
        pltpu.sync_copy(edst_h.at[pl.ds(base, EPS)], dstv)
        pltpu.sync_copy(esrc_h.at[pl.ds(base, EPS)], srcv)
        pltpu.sync_copy(sdst_h.at[pl.ds(base, EPS)], sdv)
        pltpu.sync_copy(ssrc_h.at[pl.ds(base, EPS)], ssv)
        pltpu.sync_copy(w_h.at[pl.ds(base, EPS)], wv)
        _zero_vmem_1d(zv, 1024)

        def fill_ones(e, _):
            ones_b[pl.ds(e * L, L)] = jnp.full((L,), 1.0, f32)
            return 0
        lax.fori_loop(0, CH // L, fill_ones, 0)

        pltpu.sync_copy(zv, acc.at[pl.ds(sid * 1024, 1024)])
        plsc.subcore_barrier()

        for c in range(NCHK):
            for (vec, off, vals) in (
                    (dstv, 0, ones_b), (srcv, NN, ones_b),
                    (sdv, 2 * NN, wv.at[pl.ds(c * CH, CH)]),
                    (ssv, 3 * NN, wv.at[pl.ds(c * CH, CH)])):
                for g in range(CH // L):
                    v = vec[pl.ds(c * CH + g * L, L)]
                    idxb[pl.ds(g * L, L)] = v + off
                pltpu.sync_copy(vals, acc.at[idxb], add=True)
        pltpu.sync_copy(zv.at[pl.ds(0, CH)], acc.at[idxb], add=True)
        pltpu.sync_copy(acc.at[pl.ds(0, L)], drb)
        plsc.subcore_barrier()
        pltpu.sync_copy(acc.at[pl.ds(sid * 1024, 1024)],
                        out_h.at[cid, pl.ds(sid * 1024, 1024)])

    fn = pl.kernel(
        body,
        out_type=jax.ShapeDtypeStruct((NC, AW), f32),
        mesh=_sc_mesh(),
        compiler_params=pltpu.CompilerParams(use_tc_tiling_on_sc=False),
        scratch_types=[
            pltpu.VMEM_SHARED((AW,), f32),
            pltpu.VMEM((EPS,), i32), pltpu.VMEM((EPS,), i32),
            pltpu.VMEM((EPS,), i32), pltpu.VMEM((EPS,), i32),
            pltpu.VMEM((EPS,), f32),
            pltpu.VMEM((CH,), i32),
            pltpu.VMEM((CH,), f32),
            pltpu.VMEM((1024,), f32),
            pltpu.VMEM((L,), f32),
        ],
    )
    return fn(edst, esrc, sdst, ssrc, w)


def _sc_build_adj(edst, esrc, tok):
    EPS = EE // NS
    CH = 128
    NCHK = EPS // CH
    SLABW = 256 * NN
    NSLAB = (NN * NN) // SLABW
    ZW = 16384

    def body(edst_h, esrc_h, tok_h, out_h,
             acc, dstv, srcv, idxb, ones_b, zv, drb):
        cid = lax.axis_index("c")
        sid = lax.axis_index("s")
        base = sid * EPS
        pltpu.sync_copy(tok_h, drb)
        pltpu.sync_copy(edst_h.at[pl.ds(base, EPS)], dstv)
        pltpu.sync_copy(esrc_h.at[pl.ds(base, EPS)], srcv)
        _zero_vmem_1d(zv, ZW)

        def fill_ones(e, _):
            ones_b[pl.ds(e * L, L)] = jnp.full((L,), 1.0, f32)
            return 0
        lax.fori_loop(0, CH // L, fill_ones, 0)

        def do_slab(t, _):
            slab = 2 * t + cid
            word0 = slab * SLABW
            for z in range(SLABW // NS // ZW):
                pltpu.sync_copy(zv, acc.at[pl.ds(sid * (SLABW // NS) + z * ZW, ZW)])
            @pl.when(sid == 0)
            def _():
                pltpu.sync_copy(zv.at[pl.ds(0, L)], acc.at[pl.ds(SLABW, L)])
            plsc.subcore_barrier()

            def do_chunk(c, _):
                for g in range(CH // L):
                    d = dstv[pl.ds(c * CH + g * L, L)]
                    s = srcv[pl.ds(c * CH + g * L, L)]
                    loc = d * NN + s - word0
                    ok = (loc >= 0) & (loc < SLABW)
                    idxb[pl.ds(g * L, L)] = jnp.where(ok, loc, SLABW)
                pltpu.sync_copy(ones_b, acc.at[idxb], add=True)
                return 0
            lax.fori_loop(0, NCHK, do_chunk, 0)
            pltpu.sync_copy(zv.at[pl.ds(0, CH)], acc.at[idxb], add=True)
            pltpu.sync_copy(acc.at[pl.ds(SLABW, L)], drb)
            plsc.subcore_barrier()
            pltpu.sync_copy(acc.at[pl.ds(sid * (SLABW // NS), SLABW // NS)],
                            out_h.at[pl.ds(word0 + sid * (SLABW // NS), SLABW // NS)])
            plsc.subcore_barrier()
            return 0
        lax.fori_loop(0, NSLAB // NC, do_slab, 0)

    fn = pl.kernel(
        body,
        out_type=jax.ShapeDtypeStruct((NN * NN,), f32),
        mesh=_sc_mesh(),
        compiler_params=pltpu.CompilerParams(use_tc_tiling_on_sc=False),
        scratch_types=[
            pltpu.VMEM_SHARED((SLABW + L,), f32),
            pltpu.VMEM((EPS,), i32), pltpu.VMEM((EPS,), i32),
            pltpu.VMEM((CH,), i32),
            pltpu.VMEM((CH,), f32),
            pltpu.VMEM((ZW,), f32),
            pltpu.VMEM((L,), f32),
        ],
    )
    return fn(edst, esrc, tok)


def _sc_prop(table2, gat, sct, w, tok):
    _, n, d2 = table2.shape
    EPS = EE // NS
    CH = 128
    NCHK = EPS // CH
    RPS = NN // NS
    weighted = w is not None

    def body(*refs):
        if weighted:
            (tab_h, gat_h, sct_h, w_h, tok_h, out_h,
             acc, gidx, sidx, rows0, rows1, zrows, drb, tkv, wv, sem) = refs
        else:
            (tab_h, gat_h, sct_h, tok_h, out_h,
             acc, gidx, sidx, rows0, rows1, zrows, drb, tkv, sem) = refs
        rows = (rows0, rows1)
        cid = lax.axis_index("c")
        sid = lax.axis_index("s")
        pltpu.sync_copy(tok_h, tkv)
        pltpu.sync_copy(gat_h.at[pl.ds(sid * NCHK, NCHK)], gidx)
        pltpu.sync_copy(sct_h.at[pl.ds(sid * NCHK, NCHK)], sidx)
        if weighted:
            pltpu.sync_copy(w_h.at[pl.ds(sid * EPS, EPS)], wv.at[pl.ds(0, EPS)])
        _zero_vmem_rows(zrows, CH, d2)
        for z in range(RPS // CH):
            pltpu.sync_copy(zrows, acc.at[pl.ds(sid * RPS + z * CH, CH)])
        plsc.subcore_barrier()
        pending = pltpu.async_copy(tab_h.at[cid].at[gidx.at[0]], rows[0], sem)
        for c in range(NCHK):
            pending.wait()
            if c + 1 < NCHK:
                pending = pltpu.async_copy(
                    tab_h.at[cid].at[gidx.at[c + 1]], rows[(c + 1) % 2], sem)
            rb = rows[c % 2]
            if weighted:
                def scale(e, _):
                    ws = wv[pl.ds(c * CH + e, L)][0]
                    for g in range(d2 // L):
                        rb[e, pl.ds(g * L, L)] = rb[e, pl.ds(g * L, L)] * ws
                    return 0
                lax.fori_loop(0, CH, scale, 0)
            pltpu.sync_copy(rb, acc.at[sidx.at[c]], add=True)
        pltpu.sync_copy(zrows, acc.at[sidx.at[NCHK - 1]], add=True)
        pltpu.sync_copy(acc.at[pl.ds(0, 1)], drb)
        plsc.subcore_barrier()
        pltpu.sync_copy(acc.at[pl.ds(sid * RPS, RPS)],
                        out_h.at[cid, pl.ds(sid * RPS, RPS)])

    scratch = [
        pltpu.VMEM_SHARED((NN, d2), f32),
        pltpu.VMEM((NCHK, CH), i32),
        pltpu.VMEM((NCHK, CH), i32),
        pltpu.VMEM((CH, d2), f32),
        pltpu.VMEM((CH, d2), f32),
        pltpu.VMEM((CH, d2), f32),
        pltpu.VMEM((1, d2), f32),
        pltpu.VMEM((L,), f32),
    ]
    if weighted:
        scratch.append(pltpu.VMEM((EPS + L,), f32))
    scratch.append(pltpu.SemaphoreType.DMA)
    fn = pl.kernel(
        body,
        out_type=jax.ShapeDtypeStruct((NC, NN, d2), f32),
        mesh=_sc_mesh(),
        compiler_params=pltpu.CompilerParams(use_tc_tiling_on_sc=False),
        scratch_types=scratch,
    )
    args = (table2, gat.reshape(-1, CH), sct.reshape(-1, CH))
    args += ((w,) if weighted else ()) + (tok,)
    return fn(*args)


def _halves(t):
    d2 = t.shape[1] // 2
    return jnp.stack([t[:, :d2], t[:, d2:]])


def _unhalves(p):
    return jnp.concatenate([p[0], p[1]], axis=1)


def _bases_2d(x, gt):
    g = [gt[i][None, :] for i in range(12)]
    B = [jnp.where((x >= g[i]) & (x < g[i + 1]), 1.0, 0.0).astype(f32)
         for i in range(11)]
    for j in range(1, 4):
        B = [(x - g[i]) / (g[i + j] - g[i]) * B[i]
             + (g[i + j + 1] - x) / (g[i + j + 1] - g[i + 1]) * B[i + 1]
             for i in range(len(B) - 1)]
    return B


def _kan_compute(x, gt, bw, sw_ref):
    silu = x * jax.nn.sigmoid(x)
    acc = lax.dot_general(silu, bw, (((1,), (1,)), ((), ())),
                          preferred_element_type=f32)
    for j, Bj in enumerate(_bases_2d(x, gt)):
        acc += lax.dot_general(Bj, sw_ref[j], (((1,), (1,)), ((), ())),
                               preferred_element_type=f32)
    return acc


def _kan(parts, pre, p, posts, blk=512):
    n, din = parts[0].shape
    dout = p['base_w'].shape[0]
    nparts, npost = len(parts), len(posts)
    has_pre = pre is not None
    gt = jnp.transpose(p['grid'])
    sw8 = jnp.transpose(p['spline_w'], (2, 0, 1))

    def kbody(*refs):
        k = 0
        x = refs[0][...]
        for t in range(1, nparts):
            x = x + refs[t][...]
        k = nparts
        if has_pre:
            x = x * refs[k][...]
            k += 1
        gt_v = refs[k][...]; k += 1
        bw = refs[k][...]; k += 1
        sw_ref = refs[k]; k += 1
        post_refs = refs[k:k + npost]; k += npost
        out_refs = refs[k:]
        h = _kan_compute(x, gt_v, bw, sw_ref)
        if npost == 0:
            out_refs[0][...] = h
        else:
            for t in range(npost):
                out_refs[t][...] = post_refs[t][...] * h

    grid = (n // blk,)
    rowspec = pl.BlockSpec((blk, din), lambda i: (i, 0))
    vecspec = pl.BlockSpec((blk, 1), lambda i: (i, 0))
    in_specs = [rowspec] * nparts
    args = list(parts)
    if has_pre:
        in_specs.append(vecspec); args.append(pre)
    in_specs += [pl.BlockSpec((12, din), lambda i: (0, 0)),
                 pl.BlockSpec((dout, din), lambda i: (0, 0)),
                 pl.BlockSpec((8, dout, din), lambda i: (0, 0, 0))]
    args += [gt, p['base_w'], sw8]
    in_specs += [vecspec] * npost
    args += list(posts)
    nout = max(npost, 1)
    out = pl.pallas_call(
        kbody, grid=grid, in_specs=in_specs,
        out_specs=[pl.BlockSpec((blk, dout), lambda i: (i, 0))] * nout,
        out_shape=[jax.ShapeDtypeStruct((n, dout), f32)] * nout,
    )(*args)
    return out


def _tc_matmul_A2(abf):
    bm = bn = bk = 512
    I, J, K = NN // bm, NN // bn, NN // bk

    def body(l_ref, r_ref, o_ref, acc):
        i, j, k = pl.program_id(0), pl.program_id(1), pl.program_id(2)
        @pl.when(k == 0)
        def _():
            acc[...] = jnp.zeros((bm, bn), f32)
        acc[...] += lax.dot_general(l_ref[...], r_ref[...],
                                    (((1,), (0,)), ((), ())),
                                    preferred_element_type=f32)
        @pl.when(k == K - 1)
        def _():
            ri = lax.broadcasted_iota(i32, (bm, bn), 0) + i * bm
            ci = lax.broadcasted_iota(i32, (bm, bn), 1) + j * bn
            o_ref[...] = jnp.where(ri == ci, 0.0, acc[...]).astype(jnp.bfloat16)

    return pl.pallas_call(
        body, grid=(I, J, K),
        in_specs=[pl.BlockSpec((bm, bk), lambda i, j, k: (i, k)),
                  pl.BlockSpec((bk, bn), lambda i, j, k: (k, j))],
        out_specs=pl.BlockSpec((bm, bn), lambda i, j, k: (i, j)),
        out_shape=jax.ShapeDtypeStruct((NN, NN), jnp.bfloat16),
        scratch_shapes=[pltpu.VMEM((bm, bn), f32)],
        compiler_params=pltpu.CompilerParams(
            dimension_semantics=("parallel", "parallel", "arbitrary")),
    )(abf, abf)


def _tc_degsum(m):
    bm = 512
    I = NN // bm

    def body(m_ref, rs_ref, cs_ref):
        blk = m_ref[...].astype(f32)
        rs_ref[...] = jnp.sum(blk, axis=1, keepdims=True)
        cs_ref[...] = jnp.sum(blk, axis=0, keepdims=True)[None]

    rs, cs = pl.pallas_call(
        body, grid=(I,),
        in_specs=[pl.BlockSpec((bm, NN), lambda i: (i, 0))],
        out_specs=[pl.BlockSpec((bm, 1), lambda i: (i, 0)),
                   pl.BlockSpec((1, 1, NN), lambda i: (i, 0, 0))],
        out_shape=[jax.ShapeDtypeStruct((NN, 1), f32),
                   jax.ShapeDtypeStruct((I, 1, NN), f32)],
    )(m)
    return rs, cs.reshape(I, NN)


def _tc_finalize_sp(degs_t):
    def body(d_ref, o_ref):
        d = d_ref[...]
        def rs_of(col):
            v = d[:, col:col + 1] + d[:, col + 4:col + 5]
            return lax.rsqrt(jnp.maximum(v, 1e-12))
        b = rs_of(0)
        a = rs_of(1)
        bs = rs_of(2)
        a_s = rs_of(3)
        o_ref[...] = jnp.concatenate([a, b, a_s, bs], axis=1)

    return pl.pallas_call(
        body,
        in_specs=[pl.BlockSpec((NN, 8), lambda: (0, 0))],
        out_specs=pl.BlockSpec((NN, 4), lambda: (0, 0)),
        out_shape=jax.ShapeDtypeStruct((NN, 4), f32),
    )(degs_t)


def _tc_finalize_dn(rowsum, colsum_t):
    nI = colsum_t.shape[1]

    def body(r_ref, c_ref, o_ref):
        rd = lax.rsqrt(jnp.maximum(r_ref[...], 1e-12))
        cs = jnp.sum(c_ref[...], axis=1, keepdims=True)
        rs = lax.rsqrt(jnp.maximum(cs, 1e-12))
        o_ref[...] = jnp.concatenate([rd, rs], axis=1)

    return pl.pallas_call(
        body,
        in_specs=[pl.BlockSpec((NN, 1), lambda: (0, 0)),
                  pl.BlockSpec((NN, nI), lambda: (0, 0))],
        out_specs=pl.BlockSpec((NN, 2), lambda: (0, 0)),
        out_shape=jax.ShapeDtypeStruct((NN, 2), f32),
    )(rowsum, colsum_t)


def _mm_prop(m, u, trans):
    n, d = u.shape
    bm, bk = 512, 512
    I, K = NN // bm, NN // bk

    def body(l_ref, r_ref, o_ref, acc):
        k = pl.program_id(1)
        @pl.when(k == 0)
        def _():
            acc[...] = jnp.zeros((bm, d), f32)
        dn = (((0,), (0,)), ((), ())) if trans else (((1,), (0,)), ((), ()))
        acc[...] += lax.dot_general(l_ref[...].astype(f32), r_ref[...], dn,
                                    preferred_element_type=f32)
        @pl.when(k == K - 1)
        def _():
            o_ref[...] = acc[...]

    if trans:
        lspec = pl.BlockSpec((bk, bm), lambda i, k: (k, i))
    else:
        lspec = pl.BlockSpec((bm, bk), lambda i, k: (i, k))
    return pl.pallas_call(
        body, grid=(I, K),
        in_specs=[lspec, pl.BlockSpec((bk, d), lambda i, k: (k, 0))],
        out_specs=pl.BlockSpec((bm, d), lambda i, k: (i, 0)),
        out_shape=jax.ShapeDtypeStruct((NN, d), f32),
        scratch_shapes=[pltpu.VMEM((bm, d), f32)],
        compiler_params=pltpu.CompilerParams(
            dimension_semantics=("parallel", "arbitrary")),
    )(m, u)


def _fu_assemble(fp, feat_cfgs, blk=512):
    d = feat_cfgs[0][0][0].shape[1]
    att = fp['att']
    npart = [len(c[0]) for c in feat_cfgs]

    def body(*refs):
        i = pl.program_id(0)
        k = 0
        feats = []
        for t in range(3):
            x = refs[k][...]
            for _ in range(1, npart[t]):
                k += 1
                x = x + refs[k][...]
            k += 1
            x = x * refs[k][...]
            k += 1
            feats.append(x)
        lng = refs[k][...]; k += 1
        lnb = refs[k][...]; k += 1
        W = refs[k][...]; k += 1
        bv = refs[k][...]; k += 1
        qv = refs[k][...]; k += 1
        F_ref, wf_ref, cs_ref = refs[k], refs[k + 1], refs[k + 2]

        logits = []
        for t in range(3):
            f = feats[t]
            m = jnp.mean(f, axis=1, keepdims=True)
            v = jnp.mean((f - m) ** 2, axis=1, keepdims=True)
            fn = (f - m) * lax.rsqrt(v + 1e-5) * lng + lnb
            tt = jnp.tanh(lax.dot_general(fn, W, (((1,), (1,)), ((), ())),
                                          preferred_element_type=f32) + bv)
            logits.append(lax.dot_general(tt, qv, (((1,), (1,)), ((), ())),
                                          preferred_element_type=f32))
        mx = jnp.maximum(jnp.maximum(logits[0], logits[1]), logits[2])
        es = [jnp.exp(lg - mx) for lg in logits]
        den = es[0] + es[1] + es[2]
        wf = (es[0] * feats[0] + es[1] * feats[1] + es[2] * feats[2]) / den
        F = jnp.concatenate(feats, axis=1)
        F_ref[...] = F
        wf_ref[...] = wf
        @pl.when(i == 0)
        def _():
            cs_ref[...] = jnp.zeros((1, 3 * d), f32)
        cs_ref[...] += jnp.sum(F, axis=0, keepdims=True)

    grid = (NN // blk,)
    rowspec = pl.BlockSpec((blk, d), lambda i: (i, 0))
    vecspec = pl.BlockSpec((blk, 1), lambda i: (i, 0))
    in_specs, args = [], []
    for parts, scale in feat_cfgs:
        in_specs += [rowspec] * len(parts) + [vecspec]
        args += list(parts) + [scale]
    in_specs += [pl.BlockSpec((1, d), lambda i: (0, 0))] * 2
    args += [att['ln_g'][None, :], att['ln_b'][None, :]]
    in_specs += [pl.BlockSpec((d, d), lambda i: (0, 0))]
    args += [att['W']]
    in_specs += [pl.BlockSpec((1, d), lambda i: (0, 0))] * 2
    args += [att['b'][None, :], att['q'][None, :]]
    return pl.pallas_call(
        body, grid=grid, in_specs=in_specs,
        out_specs=[pl.BlockSpec((blk, 3 * d), lambda i: (i, 0)),
                   pl.BlockSpec((blk, d), lambda i: (i, 0)),
                   pl.BlockSpec((1, 3 * d), lambda i: (0, 0))],
        out_shape=[jax.ShapeDtypeStruct((NN, 3 * d), f32),
                   jax.ShapeDtypeStruct((NN, d), f32),
                   jax.ShapeDtypeStruct((1, 3 * d), f32)],
    )(*args)


def _fu_sumsq(F, cs, blk=512):
    dcols = F.shape[1]

    def body(f_ref, c_ref, o_ref):
        i = pl.program_id(0)
        m = c_ref[...] * (1.0 / NN)
        dev = f_ref[...] - m
        @pl.when(i == 0)
        def _():
            o_ref[...] = jnp.zeros((1, dcols), f32)
        o_ref[...] += jnp.sum(dev * dev, axis=0, keepdims=True)

    return pl.pallas_call(
        body, grid=(NN // blk,),
        in_specs=[pl.BlockSpec((blk, dcols), lambda i: (i, 0)),
                  pl.BlockSpec((1, dcols), lambda i: (0, 0))],
        out_specs=pl.BlockSpec((1, dcols), lambda i: (0, 0)),
        out_shape=jax.ShapeDtypeStruct((1, dcols), f32),
    )(F, cs)


def _fu_kan(F, cs, ss, fp, blk=256):
    dcols = F.shape[1]
    p = fp['kan']
    dout = p['base_w'].shape[0]
    gt = jnp.transpose(p['grid'])
    sw8 = jnp.transpose(p['spline_w'], (2, 0, 1))

    def body(f_ref, c_ref, s_ref, g_ref, b_ref, gt_ref, bw_ref, sw_ref,
             o_ref, oc_ref):
        i = pl.program_id(0)
        m = c_ref[...] * (1.0 / NN)
        v = s_ref[...] * (1.0 / NN)
        xb = (f_ref[...] - m) * lax.rsqrt(v + 1e-5) * g_ref[...] + b_ref[...]
        h = _kan_compute(xb, gt_ref[...], bw_ref[...], sw_ref)
        o_ref[...] = h
        @pl.when(i == 0)
        def _():
            oc_ref[...] = jnp.zeros((1, dout), f32)
        oc_ref[...] += jnp.sum(h, axis=0, keepdims=True)

    cspec = pl.BlockSpec((1, dcols), lambda i: (0, 0))
    return pl.pallas_call(
        body, grid=(NN // blk,),
        in_specs=[pl.BlockSpec((blk, dcols), lambda i: (i, 0)), cspec, cspec,
                  cspec, cspec,
                  pl.BlockSpec((12, dcols), lambda i: (0, 0)),
                  pl.BlockSpec((dout, dcols), lambda i: (0, 0)),
                  pl.BlockSpec((8, dout, dcols), lambda i: (0, 0, 0))],
        out_specs=[pl.BlockSpec((blk, dout), lambda i: (i, 0)),
                   pl.BlockSpec((1, dout), lambda i: (0, 0))],
        out_shape=[jax.ShapeDtypeStruct((NN, dout), f32),
                   jax.ShapeDtypeStruct((1, dout), f32)],
    )(F, cs, ss, fp['bn1_g'][None, :], fp['bn1_b'][None, :], gt,
      p['base_w'], sw8)


def _fu_final(wf, kout, kcs, kss, fp, blk=512):
    d = kout.shape[1]

    def body(w_ref, k_ref, c_ref, s_ref, g_ref, b_ref, o_ref):
        m = c_ref[...] * (1.0 / NN)
        v = s_ref[...] * (1.0 / NN)
        h = (k_ref[...] - m) * lax.rsqrt(v + 1e-5) * g_ref[...] + b_ref[...]
        o_ref[...] = jnp.concatenate([w_ref[...], h], axis=1)

    cspec = pl.BlockSpec((1, d), lambda i: (0, 0))
    return pl.pallas_call(
        body, grid=(NN // blk,),
        in_specs=[pl.BlockSpec((blk, d), lambda i: (i, 0)),
                  pl.BlockSpec((blk, d), lambda i: (i, 0)),
                  cspec, cspec, cspec, cspec],
        out_specs=pl.BlockSpec((blk, 2 * d), lambda i: (i, 0)),
        out_shape=jax.ShapeDtypeStruct((NN, 2 * d), f32),
    )(wf, kout, kcs, kss, fp['bn2_g'][None, :], fp['bn2_b'][None, :])


def _fusion(fp, f1, f2, f3):
    F, wf, cs = _fu_assemble(fp, [f1, f2, f3])
    ss = _fu_sumsq(F, cs)
    kout, kcs = _fu_kan(F, cs, ss, fp)
    kss = _fu_sumsq(kout, kcs)
    return _fu_final(wf, kout, kcs, kss, fp)


def kernel(x, edge_index, sim_index, sim_weight, params):
    src, dst = edge_index[0], edge_index[1]
    ssrc, sdst = sim_index[0], sim_index[1]

    degs = _sc_degrees(dst, src, sdst, ssrc, sim_weight)
    degs_t = jnp.transpose(degs.reshape(8, NN))
    Aflat = _sc_build_adj(dst, src, degs[0, :L])
    A = Aflat.reshape(NN, NN)
    A2 = _tc_matmul_A2(A.astype(jnp.bfloat16))
    rowsum, colsum_part = _tc_degsum(A2)
    scales = _tc_finalize_sp(degs_t)
    a_ = scales[:, 0:1]
    b_ = scales[:, 1:2]
    as_ = scales[:, 2:3]
    bs_ = scales[:, 3:4]
    scales2 = _tc_finalize_dn(rowsum, jnp.transpose(colsum_part))
    rd = scales2[:, 0:1]
    rs = scales2[:, 1:2]

    pdd, pco, psm = params['ddi'], params['co'], params['sim']
    t_in, t_out = _kan([x], None, pdd['kan1'], [a_, b_])
    t_sim = _kan([x], None, psm['kan1'], [as_])[0]
    u_in, u_out = _kan([x], None, pco['kan1'], [rs, rd])

    P_in = _sc_prop(_halves(t_in), src, dst, None, Aflat[:L])
    P_out = _sc_prop(_halves(t_out), dst, src, None, P_in[0, 0, :L])
    S1 = _sc_prop(_halves(t_sim), ssrc, sdst, sim_weight, P_out[0, 0, :L])
    Y_in = _mm_prop(A2, u_in, False)
    Y_out = _mm_prop(A2, u_out, True)

    t2_in, = _kan([_unhalves(P_in)], b_, pdd['kan2'], [a_])
    t2_out, = _kan([_unhalves(P_out)], a_, pdd['kan2'], [b_])
    t2_sim, = _kan([_unhalves(S1)], bs_, psm['kan2'], [as_])
    u2_in, = _kan([Y_in], rd, pco['kan2'], [rs])
    u2_out, = _kan([Y_out], rs, pco['kan2'], [rd])

    X_in = _sc_prop(_halves(t2_in), src, dst, None, S1[0, 0, :L])
    X_out = _sc_prop(_halves(t2_out), dst, src, None, X_in[0, 0, :L])
    S2 = _sc_prop(_halves(t2_sim), ssrc, sdst, sim_weight, X_out[0, 0, :L])
    Y2_in = _mm_prop(A2, u2_in, False)
    Y2_out = _mm_prop(A2, u2_out, True)

    x_sim_f = _unhalves(S2)
    x_in = _fusion(params['in_fusion'],
                   ([_unhalves(X_in)], b_), ([Y2_in], rd), ([x_sim_f], bs_))
    x_out = _fusion(params['out_fusion'],
                    ([_unhalves(X_out)], a_), ([Y2_out], rs), ([x_sim_f], bs_))
    return (x_in, x_out)
